# Initial kernel scaffold; baseline (speedup 1.0000x reference)
import jax
import jax.numpy as jnp
from jax.experimental import pallas as pl


def kernel(xh_atoms, xh_residues, t, mask_atoms, mask_residues, edge_index, edge_types, params):
    raise NotImplementedError("write your pallas kernel here")


# speedup vs baseline: 1.4011x; 1.4011x over previous
import functools
import numpy as np
import jax
import jax.numpy as jnp
from jax import lax
from jax.experimental import pallas as pl
from jax.experimental.pallas import tpu as pltpu
from jax.experimental.pallas import tpu_sc as plsc

ATOM_NF = 16; RES_NF = 21; HID = 128; NRBF = 32; CUTOFF = 5.0
NLAYERS = 4; NGRAPH = 32
N_AT = 5000; N_RES = 5000; NN = 10000; NE = 160000
NP = 10240
EP = 163840
NCORE = 2; NSUB = 16
KB = 128
ET = EP // NSUB
NBLK = ET // KB
RPT = NP // NSUB
EPT = EP // (NCORE * NSUB)
RB = 256
NRB = NP // RB

_f32 = jnp.float32
_i32 = jnp.int32

_ALPHA = 5.0 / CUTOFF
_MEANS = np.linspace(float(np.exp(-CUTOFF)), 1.0, NRBF).astype(np.float32)
_BETA = float(((2.0 / NRBF) * (1.0 - np.exp(-CUTOFF))) ** -2)


def _silu(x):
    return x * (1.0 / (1.0 + jnp.exp(-x)))


def _prologue_body(xh_ref, w1_ref, ab1_ref, rb1_ref, w2a_ref, ab2_ref,
                   w2r_ref, rb2_ref, win0_ref, win1_ref, t_ref, h0_ref):
    i = pl.program_id(0)
    x = xh_ref[...]
    rows = i * RB + lax.broadcasted_iota(_i32, (RB, 1), 0)
    is_atom = rows < N_AT
    b1 = jnp.where(is_atom, ab1_ref[...], rb1_ref[...])
    h1 = _silu(jnp.dot(x, w1_ref[...], preferred_element_type=_f32) + b1)
    h2a = jnp.dot(h1, w2a_ref[...], preferred_element_type=_f32) + ab2_ref[...]
    h2r = jnp.dot(h1, w2r_ref[...], preferred_element_type=_f32) + rb2_ref[...]
    h2 = jnp.where(is_atom, h2a, h2r)
    h0 = jnp.dot(h2, win0_ref[...], preferred_element_type=_f32)
    h0_ref[...] = h0 + t_ref[0, 0] * win1_ref[...]


def _prologue(xh_all, w1c, ab1, rb1, w2a, ab2, w2r, rb2, win0, win1, tb):
    whole = lambda shape: pl.BlockSpec(shape, lambda i: (0,) * len(shape))
    return pl.pallas_call(
        _prologue_body,
        grid=(NRB,),
        in_specs=[pl.BlockSpec((RB, 128), lambda i: (i, 0)),
                  whole((128, 128)), whole((1, 128)), whole((1, 128)),
                  whole((128, 128)), whole((1, 128)),
                  whole((128, 128)), whole((1, 128)),
                  whole((128, 128)), whole((1, 128)), whole((1, 128))],
        out_specs=pl.BlockSpec((RB, 128), lambda i: (i, 0)),
        out_shape=jax.ShapeDtypeStruct((NP, 128), _f32),
    )(xh_all, w1c, ab1, rb1, w2a, ab2, w2r, rb2, win0, win1, tb)


def _geom_body(px_hbm, py_hbm, pz_hbm, src_hbm, dst_hbm,
               dx_hbm, dy_hbm, dz_hbm, d2_hbm,
               xs, ys, zs, sbuf, dbuf, ox, oy, oz, o2):
    cid = lax.axis_index("c")
    sid = lax.axis_index("s")
    wid = cid * NSUB + sid
    base = wid * EPT
    pltpu.sync_copy(px_hbm, xs)
    pltpu.sync_copy(py_hbm, ys)
    pltpu.sync_copy(pz_hbm, zs)
    pltpu.sync_copy(src_hbm.at[pl.ds(base, EPT)], sbuf)
    pltpu.sync_copy(dst_hbm.at[pl.ds(base, EPT)], dbuf)

    def body(g, _):
        s16 = sbuf[pl.ds(g * 16, 16)]
        d16 = dbuf[pl.ds(g * 16, 16)]
        dxv = plsc.load_gather(xs, [d16]) - plsc.load_gather(xs, [s16])
        dyv = plsc.load_gather(ys, [d16]) - plsc.load_gather(ys, [s16])
        dzv = plsc.load_gather(zs, [d16]) - plsc.load_gather(zs, [s16])
        d2v = dxv * dxv + dyv * dyv + dzv * dzv
        ox[pl.ds(g * 16, 16)] = dxv
        oy[pl.ds(g * 16, 16)] = dyv
        oz[pl.ds(g * 16, 16)] = dzv
        o2[pl.ds(g * 16, 16)] = d2v
        return 0

    lax.fori_loop(0, EPT // 16, body, 0)
    pltpu.sync_copy(ox, dx_hbm.at[pl.ds(base, EPT)])
    pltpu.sync_copy(oy, dy_hbm.at[pl.ds(base, EPT)])
    pltpu.sync_copy(oz, dz_hbm.at[pl.ds(base, EPT)])
    pltpu.sync_copy(o2, d2_hbm.at[pl.ds(base, EPT)])


def _geom(px, py, pz, src, dst):
    mesh = plsc.VectorSubcoreMesh(core_axis_name="c", subcore_axis_name="s", num_cores=NCORE, num_subcores=NSUB)
    out = jax.ShapeDtypeStruct((EP,), _f32)
    fn = pl.kernel(
        _geom_body,
        out_type=(out, out, out, out),
        mesh=mesh,
        compiler_params=pltpu.CompilerParams(needs_layout_passes=False),
        scratch_types=[pltpu.VMEM((NP,), _f32)] * 3
        + [pltpu.VMEM((EPT,), _i32)] * 2
        + [pltpu.VMEM((EPT,), _f32)] * 4,
    )
    return fn(px, py, pz, src, dst)


def _feat_body(dx_ref, dy_ref, dz_ref, d2_ref, et_ref, wrbf_ref, ete_ref,
               ux_ref, uy_ref, uz_ref, filt_ref):
    i = pl.program_id(0)
    d2 = d2_ref[...]
    dist = jnp.sqrt(d2 + 1e-12)
    inv = 1.0 / (dist + 1e-8)
    ux_ref[...] = dx_ref[...] * inv
    uy_ref[...] = dy_ref[...] * inv
    uz_ref[...] = dz_ref[...] * inv
    ed = jnp.exp(-_ALPHA * dist)
    env = 0.5 * (jnp.cos(jnp.pi * jnp.clip(dist, 0.0, CUTOFF) / CUTOFF) + 1.0)
    m0 = float(np.exp(-CUTOFF))
    means = (m0 + lax.broadcasted_iota(_i32, (NRBF, 1), 0).astype(_f32)
             * ((1.0 - m0) / (NRBF - 1)))
    eidx = (i * 1024 + lax.broadcasted_iota(_i32, (8, 128), 0) * 128
            + lax.broadcasted_iota(_i32, (8, 128), 1))
    valid = eidx < NE
    envm = jnp.where(valid, env, 0.0)
    et = jnp.where(valid, et_ref[...], -1)
    g3 = lax.broadcasted_iota(_i32, (3, 1), 0)
    parts = []
    for r in range(8):
        edr = ed[r:r + 1]
        rbf = jnp.exp(-_BETA * (edr - means) ** 2) * envm[r:r + 1]
        ohf = (g3 == et[r:r + 1]).astype(_f32)
        parts.append(jnp.concatenate(
            [rbf, ohf, jnp.zeros((13, 128), _f32)], axis=0))
    feat = jnp.concatenate(parts, axis=1)
    for l in range(NLAYERS):
        wr = wrbf_ref[l]
        etew = jnp.dot(ete_ref[...], wr, preferred_element_type=_f32)
        w = jnp.concatenate([wr, etew[0:3], jnp.zeros((13, 128), _f32)], axis=0)
        ft = lax.dot_general(w, feat, (((0,), (0,)), ((), ())),
                             preferred_element_type=_f32)
        filt_ref[l] = _silu(ft)


def _features(dxr, dyr, dzr, d2r, et2d, wrbf, ete8):
    row = pl.BlockSpec((8, 128), lambda i: (i, 0))
    outr = jax.ShapeDtypeStruct((EP // 128, 128), _f32)
    return pl.pallas_call(
        _feat_body,
        grid=(EP // 1024,),
        in_specs=[row, row, row, row, row,
                  pl.BlockSpec((NLAYERS, NRBF, 128), lambda i: (0, 0, 0)),
                  pl.BlockSpec((8, 32), lambda i: (0, 0))],
        out_specs=[row, row, row,
                   pl.BlockSpec((NLAYERS, 128, 1024), lambda i: (0, 0, i))],
        out_shape=[outr, outr, outr,
                   jax.ShapeDtypeStruct((NLAYERS, 128, EP), _f32)],
    )(dxr, dyr, dzr, d2r, et2d, wrbf, ete8)


def _hw_body(h0_ref, wh_ref, out_ref):
    out_ref[...] = jnp.dot(h0_ref[...], wh_ref[...],
                           preferred_element_type=_f32)


def _hw(h0, wh):
    return pl.pallas_call(
        _hw_body,
        grid=(NRB,),
        in_specs=[pl.BlockSpec((RB, 128), lambda i: (i, 0)),
                  pl.BlockSpec((128, 128), lambda i: (0, 0))],
        out_specs=pl.BlockSpec((RB, 128), lambda i: (i, 0)),
        out_shape=jax.ShapeDtypeStruct((NP, 128), _f32),
    )(h0, wh)


def _msg_body(hw_hbm, filt_hbm, src_hbm, dst_hbm, ux_hbm, uy_hbm, uz_hbm,
              out_hbm, acc, src_blk, dst0, dst1, uxb, uyb, uzb,
              fbuf, hbuf, obuf, dsem):
    cid = lax.axis_index("c")
    sid = lax.axis_index("s")
    ebase = sid * ET

    def zero_hbuf(r, _):
        for q in range(8):
            hbuf[r, pl.ds(q * 16, 16)] = jnp.zeros((16,), _f32)
        return 0

    for p in range(2):
        c = cid * 2 + p
        lax.fori_loop(0, 32, zero_hbuf, 0)

        def zero_acc(q, _):
            pltpu.sync_copy(hbuf, acc.at[pl.ds(sid * RPT + q * 32, 32)])
            return 0
        lax.fori_loop(0, RPT // 32, zero_acc, 0)
        plsc.subcore_barrier()

        def block(b, _):
            e0 = ebase + b * KB
            ds_ = [pltpu.async_copy(src_hbm.at[pl.ds(e0, KB)], src_blk, dsem),
                   pltpu.async_copy(dst_hbm.at[pl.ds(e0, 64)], dst0, dsem),
                   pltpu.async_copy(dst_hbm.at[pl.ds(e0 + 64, 64)], dst1, dsem),
                   pltpu.async_copy(ux_hbm.at[pl.ds(e0, KB)], uxb, dsem),
                   pltpu.async_copy(uy_hbm.at[pl.ds(e0, KB)], uyb, dsem),
                   pltpu.async_copy(uz_hbm.at[pl.ds(e0, KB)], uzb, dsem),
                   pltpu.async_copy(
                       filt_hbm.at[pl.ds(c * 32, 32), pl.ds(e0, KB)],
                       fbuf, dsem)]
            for d in ds_:
                d.wait()
            for half in range(2):
                for qq in range(2):
                    q = half * 2 + qq
                    pltpu.sync_copy(
                        hw_hbm.at[src_blk.at[pl.ds(q * 32, 32)]], hbuf)
                    for g2 in range(2):
                        be = q * 32 + g2 * 16
                        vx = uxb[pl.ds(be, 16)]
                        vy = uyb[pl.ds(be, 16)]
                        vz = uzb[pl.ds(be, 16)]
                        el_h = lax.iota(_i32, 16) + g2 * 16
                        el_o = lax.iota(_i32, 16) + (qq * 32 + g2 * 16)
                        hbase = jnp.full((16,), 32, _i32) * c
                        for h in range(32):
                            f = fbuf[h, pl.ds(be, 16)]
                            w = plsc.load_gather(hbuf, [el_h, hbase + h])
                            m = f * w
                            plsc.store_scatter(
                                obuf, [el_o, jnp.full((16,), h, _i32)], m)
                            plsc.store_scatter(
                                obuf, [el_o, jnp.full((16,), 32 + h, _i32)],
                                m * vx)
                            plsc.store_scatter(
                                obuf, [el_o, jnp.full((16,), 64 + h, _i32)],
                                m * vy)
                            plsc.store_scatter(
                                obuf, [el_o, jnp.full((16,), 96 + h, _i32)],
                                m * vz)
                dref = dst0 if half == 0 else dst1
                pltpu.sync_copy(obuf, acc.at[dref], add=True)
            return 0

        lax.fori_loop(0, NBLK, block, 0)
        plsc.subcore_barrier()
        pltpu.sync_copy(acc.at[pl.ds(sid * RPT, RPT)],
                        out_hbm.at[pl.ds(c * NP + sid * RPT, RPT)])
        plsc.subcore_barrier()


def _msg(hw, filt_l, src, dst, ux, uy, uz):
    mesh = plsc.VectorSubcoreMesh(core_axis_name="c", subcore_axis_name="s", num_cores=NCORE, num_subcores=NSUB)
    fn = pl.kernel(
        _msg_body,
        out_type=jax.ShapeDtypeStruct((4 * NP, 128), _f32),
        mesh=mesh,
        compiler_params=pltpu.CompilerParams(needs_layout_passes=False),
        scratch_types=[
            pltpu.VMEM_SHARED((NP, 128), _f32),
            pltpu.VMEM((KB,), _i32),
            pltpu.VMEM((64,), _i32),
            pltpu.VMEM((64,), _i32),
            pltpu.VMEM((KB,), _f32),
            pltpu.VMEM((KB,), _f32),
            pltpu.VMEM((KB,), _f32),
            pltpu.VMEM((32, KB), _f32),
            pltpu.VMEM((32, 128), _f32),
            pltpu.VMEM((64, 128), _f32),
            pltpu.SemaphoreType.DMA,
        ],
    )
    return fn(hw, filt_l, src, dst, ux, uy, uz)


def _update_body(h0_ref, sco_ref, vec_ref, wv_ref, h0o_ref, veco_ref):
    s = sco_ref[...]
    agg = jnp.concatenate([s[k, :, 0:32] for k in range(4)], axis=1)
    h0o_ref[...] = h0_ref[...] + _silu(agg)
    wv = wv_ref[...]
    for ci in range(3):
        c0 = 32 * (ci + 1)
        d = jnp.concatenate([s[k, :, c0:c0 + 32] for k in range(4)], axis=1)
        veco_ref[ci] = jnp.dot(vec_ref[ci] + d, wv, preferred_element_type=_f32)


def _update(h0, sco4, vec3, wv):
    return pl.pallas_call(
        _update_body,
        grid=(NRB,),
        in_specs=[pl.BlockSpec((RB, 128), lambda i: (i, 0)),
                  pl.BlockSpec((4, RB, 128), lambda i: (0, i, 0)),
                  pl.BlockSpec((3, RB, 128), lambda i: (0, i, 0)),
                  pl.BlockSpec((128, 128), lambda i: (0, 0))],
        out_specs=[pl.BlockSpec((RB, 128), lambda i: (i, 0)),
                   pl.BlockSpec((3, RB, 128), lambda i: (0, i, 0))],
        out_shape=[jax.ShapeDtypeStruct((NP, 128), _f32),
                   jax.ShapeDtypeStruct((3, NP, 128), _f32)],
    )(h0, sco4, vec3, wv)


def _vel_body(vec_ref, w1_ref, b1_ref, w2_ref, b2_ref, bt_ref,
              velp_ref, sums_ref):
    i = pl.program_id(0)
    cols = []
    w2row = w2_ref[...]
    for ci in range(3):
        sv = _silu(jnp.dot(vec_ref[ci], w1_ref[...],
                           preferred_element_type=_f32) + b1_ref[...])
        r = jnp.sum(sv * w2row, axis=1, keepdims=True) + b2_ref[0, 0]
        cols.append(r)
    rows = i * RB + lax.broadcasted_iota(_i32, (RB, 1), 0)
    ones = jnp.where(rows < NN, 1.0, 0.0)
    velp = jnp.concatenate(cols + [ones, jnp.zeros((RB, 124), _f32)], axis=1)
    velp_ref[...] = velp
    bt = bt_ref[...]
    oh = (lax.broadcasted_iota(_i32, (NGRAPH, 1), 0) == bt).astype(_f32)
    contrib = jnp.dot(oh, velp, preferred_element_type=_f32)

    @pl.when(i == 0)
    def _():
        sums_ref[...] = jnp.zeros((NGRAPH, 128), _f32)

    sums_ref[...] += contrib


def _vel(vec3, vvw1, vvb1, vvw2r, vvb2b, batch):
    return pl.pallas_call(
        _vel_body,
        grid=(NRB,),
        in_specs=[pl.BlockSpec((3, RB, 128), lambda i: (0, i, 0)),
                  pl.BlockSpec((128, 64), lambda i: (0, 0)),
                  pl.BlockSpec((1, 64), lambda i: (0, 0)),
                  pl.BlockSpec((1, 64), lambda i: (0, 0)),
                  pl.BlockSpec((1, 128), lambda i: (0, 0)),
                  pl.BlockSpec((RB,), lambda i: (i,))],
        out_specs=[pl.BlockSpec((RB, 128), lambda i: (i, 0)),
                   pl.BlockSpec((NGRAPH, 128), lambda i: (0, 0))],
        out_shape=[jax.ShapeDtypeStruct((NP, 128), _f32),
                   jax.ShapeDtypeStruct((NGRAPH, 128), _f32)],
    )(vec3, vvw1, vvb1, vvw2r, vvb2b, batch)


def _dec_body(velp_ref, sums_ref, bt_ref, h0_ref, aw1_ref, ab1_ref, aw2_ref,
              ab2_ref, rw1_ref, rb1_ref, rw2_ref, rb2_ref, out_ref):
    i = pl.program_id(0)
    s = sums_ref[...]
    cnt = s[:, 3:4]
    mean = s * (1.0 / jnp.maximum(cnt, 1.0))
    bt = bt_ref[...]
    oh = (lax.broadcasted_iota(_i32, (NGRAPH, 1), 0) == bt).astype(_f32)
    meanrows = lax.dot_general(oh, mean, (((0,), (0,)), ((), ())),
                               preferred_element_type=_f32)
    vel = velp_ref[...] - meanrows
    h = h0_ref[...]
    ha = jnp.dot(_silu(jnp.dot(h, aw1_ref[...], preferred_element_type=_f32)
                       + ab1_ref[...]), aw2_ref[...],
                 preferred_element_type=_f32) + ab2_ref[...]
    hr = jnp.dot(_silu(jnp.dot(h, rw1_ref[...], preferred_element_type=_f32)
                       + rb1_ref[...]), rw2_ref[...],
                 preferred_element_type=_f32) + rb2_ref[...]
    rows = i * RB + lax.broadcasted_iota(_i32, (RB, 1), 0)
    hf = jnp.where(rows < N_AT, ha, hr)
    out_ref[...] = jnp.concatenate(
        [vel[:, 0:3], hf[:, 0:21], jnp.zeros((RB, 8), _f32)], axis=1)


def _decode(velp, sums, batch, h0, aw1, ab1, aw2p, ab2p, rw1, rb1, rw2p, rb2p):
    whole = lambda shape: pl.BlockSpec(shape, lambda i: (0,) * len(shape))
    return pl.pallas_call(
        _dec_body,
        grid=(NRB,),
        in_specs=[pl.BlockSpec((RB, 128), lambda i: (i, 0)),
                  whole((NGRAPH, 128)),
                  pl.BlockSpec((RB,), lambda i: (i,)),
                  pl.BlockSpec((RB, 128), lambda i: (i, 0)),
                  whole((128, 128)), whole((1, 128)), whole((128, 32)),
                  whole((1, 32)),
                  whole((128, 128)), whole((1, 128)), whole((128, 32)),
                  whole((1, 32))],
        out_specs=pl.BlockSpec((RB, 32), lambda i: (i, 0)),
        out_shape=jax.ShapeDtypeStruct((NP, 32), _f32),
    )(velp, sums, batch, h0, aw1, ab1, aw2p, ab2p, rw1, rb1, rw2p, rb2p)


def kernel(xh_atoms, xh_residues, t, mask_atoms, mask_residues,
           edge_index, edge_types, params):
    p = params
    xa = xh_atoms.astype(_f32)
    xr = xh_residues.astype(_f32)

    xh_all = jnp.zeros((NP, 128), _f32)
    xh_all = xh_all.at[:N_AT, 0:ATOM_NF].set(xa[:, 3:])
    xh_all = xh_all.at[N_AT:NN, ATOM_NF:ATOM_NF + RES_NF].set(xr[:, 3:])
    pcat = jnp.concatenate([xa[:, :3], xr[:, :3],
                            jnp.zeros((NP - NN, 3), _f32)], axis=0)
    px = pcat[:, 0]; py = pcat[:, 1]; pz = pcat[:, 2]

    src = jnp.zeros((EP,), _i32).at[:NE].set(edge_index[0].astype(_i32))
    dst = jnp.zeros((EP,), _i32).at[:NE].set(edge_index[1].astype(_i32))
    et2d = (jnp.zeros((EP,), _i32).at[:NE].set(edge_types.astype(_i32))
            .reshape(EP // 128, 128))
    batch = (jnp.full((NP,), -1, _i32)
             .at[:N_AT].set(mask_atoms.astype(_i32))
             .at[N_AT:NN].set(mask_residues.astype(_i32)))

    w1c = (jnp.zeros((128, 128), _f32)
           .at[0:ATOM_NF].set(p['ae_w1'])
           .at[ATOM_NF:ATOM_NF + RES_NF].set(p['re_w1']))
    ab1 = p['ae_b1'].reshape(1, 128); rb1 = p['re_b1'].reshape(1, 128)
    ab2 = p['ae_b2'].reshape(1, 128); rb2 = p['re_b2'].reshape(1, 128)
    win0 = p['win'][:128]
    win1 = p['win'][128:129]
    tb = jnp.broadcast_to(t.reshape(1, 1).astype(_f32), (1, 128))
    ete8 = jnp.zeros((8, 32), _f32).at[0:3].set(p['ete'])
    vvw1 = p['vv_w1']
    vvb1 = p['vv_b1'].reshape(1, 64)
    vvw2r = p['vv_w2'].reshape(1, 64)
    vvb2b = jnp.broadcast_to(p['vv_b2'].reshape(1, 1), (1, 128))
    aw2p = jnp.zeros((128, 32), _f32).at[:, 0:ATOM_NF].set(p['ad_w2'])
    ab2p = jnp.zeros((1, 32), _f32).at[0, 0:ATOM_NF].set(p['ad_b2'])
    rw2p = jnp.zeros((128, 32), _f32).at[:, 0:RES_NF].set(p['rd_w2'])
    rb2p = jnp.zeros((1, 32), _f32).at[0, 0:RES_NF].set(p['rd_b2'])

    h0 = _prologue(xh_all, w1c, ab1, rb1, p['ae_w2'], ab2, p['re_w2'], rb2,
                   win0, win1, tb)
    dxr, dyr, dzr, d2r = _geom(px, py, pz, src, dst)
    ux, uy, uz, filtT = _features(
        dxr.reshape(EP // 128, 128), dyr.reshape(EP // 128, 128),
        dzr.reshape(EP // 128, 128), d2r.reshape(EP // 128, 128),
        et2d, p['mp_wrbf'], ete8)
    ux = ux.reshape(EP); uy = uy.reshape(EP); uz = uz.reshape(EP)

    vec3 = jnp.zeros((3, NP, 128), _f32)
    for l in range(NLAYERS):
        hw = _hw(h0, p['mp_wh'][l])
        sco = _msg(hw, filtT[l], src, dst, ux, uy, uz)
        h0, vec3 = _update(h0, sco.reshape(4, NP, 128), vec3, p['mp_wvec'][l])

    velp, sums = _vel(vec3, vvw1, vvb1, vvw2r, vvb2b, batch)
    res = _decode(velp, sums, batch, h0,
                  p['ad_w1'], p['ad_b1'].reshape(1, 128), aw2p, ab2p,
                  p['rd_w1'], p['rd_b1'].reshape(1, 128), rw2p, rb2p)
    atoms_output = res[:N_AT, 0:3 + ATOM_NF]
    residues_output = res[N_AT:NN, 0:3 + RES_NF]
    return (atoms_output, residues_output)


# speedup vs baseline: 1.6920x; 1.2076x over previous
import functools
import numpy as np
import jax
import jax.numpy as jnp
from jax import lax
from jax.experimental import pallas as pl
from jax.experimental.pallas import tpu as pltpu
from jax.experimental.pallas import tpu_sc as plsc

ATOM_NF = 16; RES_NF = 21; HID = 128; NRBF = 32; CUTOFF = 5.0
NLAYERS = 4; NGRAPH = 32
N_AT = 5000; N_RES = 5000; NN = 10000; NE = 160000
NP = 10240
EP = 163840
NCORE = 2; NSUB = 16
KB = 128
ET = EP // NSUB
NBLK = ET // KB
RPT = NP // NSUB
EPT = EP // (NCORE * NSUB)
RB = 256
NRB = NP // RB

_f32 = jnp.float32
_i32 = jnp.int32

_ALPHA = 5.0 / CUTOFF
_MEANS = np.linspace(float(np.exp(-CUTOFF)), 1.0, NRBF).astype(np.float32)
_BETA = float(((2.0 / NRBF) * (1.0 - np.exp(-CUTOFF))) ** -2)


def _silu(x):
    return x * (1.0 / (1.0 + jnp.exp(-x)))


def _prologue_body(xh_ref, w1_ref, ab1_ref, rb1_ref, w2a_ref, ab2_ref,
                   w2r_ref, rb2_ref, win0_ref, win1_ref, t_ref, h0_ref):
    i = pl.program_id(0)
    x = xh_ref[...]
    rows = i * RB + lax.broadcasted_iota(_i32, (RB, 1), 0)
    is_atom = rows < N_AT
    b1 = jnp.where(is_atom, ab1_ref[...], rb1_ref[...])
    h1 = _silu(jnp.dot(x, w1_ref[...], preferred_element_type=_f32) + b1)
    h2a = jnp.dot(h1, w2a_ref[...], preferred_element_type=_f32) + ab2_ref[...]
    h2r = jnp.dot(h1, w2r_ref[...], preferred_element_type=_f32) + rb2_ref[...]
    h2 = jnp.where(is_atom, h2a, h2r)
    h0 = jnp.dot(h2, win0_ref[...], preferred_element_type=_f32)
    h0_ref[...] = h0 + t_ref[0, 0] * win1_ref[...]


def _prologue(xh_all, w1c, ab1, rb1, w2a, ab2, w2r, rb2, win0, win1, tb):
    whole = lambda shape: pl.BlockSpec(shape, lambda i: (0,) * len(shape))
    return pl.pallas_call(
        _prologue_body,
        grid=(NRB,),
        in_specs=[pl.BlockSpec((RB, 128), lambda i: (i, 0)),
                  whole((128, 128)), whole((1, 128)), whole((1, 128)),
                  whole((128, 128)), whole((1, 128)),
                  whole((128, 128)), whole((1, 128)),
                  whole((128, 128)), whole((1, 128)), whole((1, 128))],
        out_specs=pl.BlockSpec((RB, 128), lambda i: (i, 0)),
        out_shape=jax.ShapeDtypeStruct((NP, 128), _f32),
    )(xh_all, w1c, ab1, rb1, w2a, ab2, w2r, rb2, win0, win1, tb)


def _geom_body(px_hbm, py_hbm, pz_hbm, src_hbm, dst_hbm,
               dx_hbm, dy_hbm, dz_hbm, d2_hbm,
               xs, ys, zs, sbuf, dbuf, ox, oy, oz, o2):
    cid = lax.axis_index("c")
    sid = lax.axis_index("s")
    wid = cid * NSUB + sid
    base = wid * EPT
    pltpu.sync_copy(px_hbm, xs)
    pltpu.sync_copy(py_hbm, ys)
    pltpu.sync_copy(pz_hbm, zs)
    pltpu.sync_copy(src_hbm.at[pl.ds(base, EPT)], sbuf)
    pltpu.sync_copy(dst_hbm.at[pl.ds(base, EPT)], dbuf)

    def body(g, _):
        s16 = sbuf[pl.ds(g * 16, 16)]
        d16 = dbuf[pl.ds(g * 16, 16)]
        dxv = plsc.load_gather(xs, [d16]) - plsc.load_gather(xs, [s16])
        dyv = plsc.load_gather(ys, [d16]) - plsc.load_gather(ys, [s16])
        dzv = plsc.load_gather(zs, [d16]) - plsc.load_gather(zs, [s16])
        d2v = dxv * dxv + dyv * dyv + dzv * dzv
        ox[pl.ds(g * 16, 16)] = dxv
        oy[pl.ds(g * 16, 16)] = dyv
        oz[pl.ds(g * 16, 16)] = dzv
        o2[pl.ds(g * 16, 16)] = d2v
        return 0

    lax.fori_loop(0, EPT // 16, body, 0)
    pltpu.sync_copy(ox, dx_hbm.at[pl.ds(base, EPT)])
    pltpu.sync_copy(oy, dy_hbm.at[pl.ds(base, EPT)])
    pltpu.sync_copy(oz, dz_hbm.at[pl.ds(base, EPT)])
    pltpu.sync_copy(o2, d2_hbm.at[pl.ds(base, EPT)])


def _geom(px, py, pz, src, dst):
    mesh = plsc.VectorSubcoreMesh(core_axis_name="c", subcore_axis_name="s", num_cores=NCORE, num_subcores=NSUB)
    out = jax.ShapeDtypeStruct((EP,), _f32)
    fn = pl.kernel(
        _geom_body,
        out_type=(out, out, out, out),
        mesh=mesh,
        compiler_params=pltpu.CompilerParams(needs_layout_passes=False),
        scratch_types=[pltpu.VMEM((NP,), _f32)] * 3
        + [pltpu.VMEM((EPT,), _i32)] * 2
        + [pltpu.VMEM((EPT,), _f32)] * 4,
    )
    return fn(px, py, pz, src, dst)


def _feat_body(dx_ref, dy_ref, dz_ref, d2_ref, et_ref, wrbf_ref, ete_ref,
               ux_ref, uy_ref, uz_ref, filt_ref):
    i = pl.program_id(0)
    d2 = d2_ref[...]
    dist = jnp.sqrt(d2 + 1e-12)
    inv = 1.0 / (dist + 1e-8)
    ux_ref[...] = dx_ref[...] * inv
    uy_ref[...] = dy_ref[...] * inv
    uz_ref[...] = dz_ref[...] * inv
    ed = jnp.exp(-_ALPHA * dist)
    env = 0.5 * (jnp.cos(jnp.pi * jnp.clip(dist, 0.0, CUTOFF) / CUTOFF) + 1.0)
    m0 = float(np.exp(-CUTOFF))
    means = (m0 + lax.broadcasted_iota(_i32, (NRBF, 1), 0).astype(_f32)
             * ((1.0 - m0) / (NRBF - 1)))
    eidx = (i * 1024 + lax.broadcasted_iota(_i32, (8, 128), 0) * 128
            + lax.broadcasted_iota(_i32, (8, 128), 1))
    valid = eidx < NE
    envm = jnp.where(valid, env, 0.0)
    et = jnp.where(valid, et_ref[...], -1)
    g3 = lax.broadcasted_iota(_i32, (3, 1), 0)
    parts = []
    for r in range(8):
        edr = ed[r:r + 1]
        rbf = jnp.exp(-_BETA * (edr - means) ** 2) * envm[r:r + 1]
        ohf = (g3 == et[r:r + 1]).astype(_f32)
        parts.append(jnp.concatenate(
            [rbf, ohf, jnp.zeros((13, 128), _f32)], axis=0))
    feat = jnp.concatenate(parts, axis=1)
    for l in range(NLAYERS):
        wr = wrbf_ref[l]
        etew = jnp.dot(ete_ref[...], wr, preferred_element_type=_f32)
        w = jnp.concatenate([wr, etew[0:3], jnp.zeros((13, 128), _f32)], axis=0)
        ft = lax.dot_general(w, feat, (((0,), (0,)), ((), ())),
                             preferred_element_type=_f32)
        filt_ref[l] = _silu(ft)


def _features(dxr, dyr, dzr, d2r, et2d, wrbf, ete8):
    row = pl.BlockSpec((8, 128), lambda i: (i, 0))
    outr = jax.ShapeDtypeStruct((EP // 128, 128), _f32)
    return pl.pallas_call(
        _feat_body,
        grid=(EP // 1024,),
        in_specs=[row, row, row, row, row,
                  pl.BlockSpec((NLAYERS, NRBF, 128), lambda i: (0, 0, 0)),
                  pl.BlockSpec((8, 32), lambda i: (0, 0))],
        out_specs=[row, row, row,
                   pl.BlockSpec((NLAYERS, 128, 1024), lambda i: (0, 0, i))],
        out_shape=[outr, outr, outr,
                   jax.ShapeDtypeStruct((NLAYERS, 128, EP), _f32)],
    )(dxr, dyr, dzr, d2r, et2d, wrbf, ete8)


def _hw_body(h0_ref, wh_ref, out_ref):
    out_ref[...] = jnp.dot(h0_ref[...], wh_ref[...],
                           preferred_element_type=_f32)


def _hw(h0, wh):
    return pl.pallas_call(
        _hw_body,
        grid=(NRB,),
        in_specs=[pl.BlockSpec((RB, 128), lambda i: (i, 0)),
                  pl.BlockSpec((128, 128), lambda i: (0, 0))],
        out_specs=pl.BlockSpec((RB, 128), lambda i: (i, 0)),
        out_shape=jax.ShapeDtypeStruct((NP, 128), _f32),
    )(h0, wh)


def _msg_body(hw_hbm, filt_hbm, sd5_hbm, out_hbm,
              acc, pbuf, dst0, dst1, fbuf, hbufa, hbufb, obuf,
              ssem, gsem):
    cid = lax.axis_index("c")
    sid = lax.axis_index("s")
    ebase = sid * ET

    def zero_hbuf(r, _):
        for q in range(8):
            hbufa[r, pl.ds(q * 16, 16)] = jnp.zeros((16,), _f32)
        return 0

    for p in range(2):
        c = cid * 2 + p
        lax.fori_loop(0, 32, zero_hbuf, 0)

        def zero_acc(q, _):
            pltpu.sync_copy(hbufa, acc.at[pl.ds(sid * RPT + q * 32, 32)])
            return 0
        lax.fori_loop(0, RPT // 32, zero_acc, 0)
        plsc.subcore_barrier()

        def block(b, _):
            e0 = ebase + b * KB
            d1 = pltpu.async_copy(sd5_hbm.at[:, pl.ds(e0, KB)], pbuf, ssem)
            d2 = pltpu.async_copy(
                filt_hbm.at[pl.ds(c * 32, 32), pl.ds(e0, KB)], fbuf, ssem)
            d1.wait()
            for g in range(4):
                dst0[pl.ds(g * 16, 16)] = pbuf[1, pl.ds(g * 16, 16)]
                dst1[pl.ds(g * 16, 16)] = pbuf[1, pl.ds(64 + g * 16, 16)]
            gds = [pltpu.async_copy(
                       hw_hbm.at[pbuf.at[0, pl.ds(0, 32)]], hbufa, gsem),
                   pltpu.async_copy(
                       hw_hbm.at[pbuf.at[0, pl.ds(32, 32)]], hbufb, gsem)]
            d2.wait()
            hbase = jnp.full((16,), 32, _i32) * c
            for q in range(4):
                gds[q].wait()
                hb = hbufa if q % 2 == 0 else hbufb
                for g2 in range(2):
                    be = q * 32 + g2 * 16
                    vx = plsc.bitcast(pbuf[2, pl.ds(be, 16)], _f32)
                    vy = plsc.bitcast(pbuf[3, pl.ds(be, 16)], _f32)
                    vz = plsc.bitcast(pbuf[4, pl.ds(be, 16)], _f32)
                    el_h = lax.iota(_i32, 16) + g2 * 16
                    el_o = lax.iota(_i32, 16) + ((q % 2) * 32 + g2 * 16)
                    for h in range(32):
                        f = fbuf[h, pl.ds(be, 16)]
                        w = plsc.load_gather(hb, [el_h, hbase + h])
                        m = f * w
                        plsc.store_scatter(
                            obuf, [el_o, jnp.full((16,), h, _i32)], m)
                        plsc.store_scatter(
                            obuf, [el_o, jnp.full((16,), 32 + h, _i32)],
                            m * vx)
                        plsc.store_scatter(
                            obuf, [el_o, jnp.full((16,), 64 + h, _i32)],
                            m * vy)
                        plsc.store_scatter(
                            obuf, [el_o, jnp.full((16,), 96 + h, _i32)],
                            m * vz)
                if q < 2:
                    gds.append(pltpu.async_copy(
                        hw_hbm.at[pbuf.at[0, pl.ds((q + 2) * 32, 32)]],
                        hbufa if q % 2 == 0 else hbufb, gsem))
                if q == 1:
                    pltpu.sync_copy(obuf, acc.at[dst0], add=True)
                if q == 3:
                    pltpu.sync_copy(obuf, acc.at[dst1], add=True)
            return 0

        lax.fori_loop(0, NBLK, block, 0)
        plsc.subcore_barrier()
        pltpu.sync_copy(acc.at[pl.ds(sid * RPT, RPT)],
                        out_hbm.at[pl.ds(c * NP + sid * RPT, RPT)])
        plsc.subcore_barrier()


def _msg(hw, filt_l, sd5):
    mesh = plsc.VectorSubcoreMesh(core_axis_name="c", subcore_axis_name="s", num_cores=NCORE, num_subcores=NSUB)
    fn = pl.kernel(
        _msg_body,
        out_type=jax.ShapeDtypeStruct((4 * NP, 128), _f32),
        mesh=mesh,
        compiler_params=pltpu.CompilerParams(needs_layout_passes=False),
        scratch_types=[
            pltpu.VMEM_SHARED((NP, 128), _f32),
            pltpu.VMEM((8, KB), _i32),
            pltpu.VMEM((64,), _i32),
            pltpu.VMEM((64,), _i32),
            pltpu.VMEM((32, KB), _f32),
            pltpu.VMEM((32, 128), _f32),
            pltpu.VMEM((32, 128), _f32),
            pltpu.VMEM((64, 128), _f32),
            pltpu.SemaphoreType.DMA,
            pltpu.SemaphoreType.DMA,
        ],
    )
    return fn(hw, filt_l, sd5)


def _update_body(h0_ref, sco_ref, vec_ref, wv_ref, h0o_ref, veco_ref):
    s = sco_ref[...]
    agg = jnp.concatenate([s[k, :, 0:32] for k in range(4)], axis=1)
    h0o_ref[...] = h0_ref[...] + _silu(agg)
    wv = wv_ref[...]
    for ci in range(3):
        c0 = 32 * (ci + 1)
        d = jnp.concatenate([s[k, :, c0:c0 + 32] for k in range(4)], axis=1)
        veco_ref[ci] = jnp.dot(vec_ref[ci] + d, wv, preferred_element_type=_f32)


def _update(h0, sco4, vec3, wv):
    return pl.pallas_call(
        _update_body,
        grid=(NRB,),
        in_specs=[pl.BlockSpec((RB, 128), lambda i: (i, 0)),
                  pl.BlockSpec((4, RB, 128), lambda i: (0, i, 0)),
                  pl.BlockSpec((3, RB, 128), lambda i: (0, i, 0)),
                  pl.BlockSpec((128, 128), lambda i: (0, 0))],
        out_specs=[pl.BlockSpec((RB, 128), lambda i: (i, 0)),
                   pl.BlockSpec((3, RB, 128), lambda i: (0, i, 0))],
        out_shape=[jax.ShapeDtypeStruct((NP, 128), _f32),
                   jax.ShapeDtypeStruct((3, NP, 128), _f32)],
    )(h0, sco4, vec3, wv)


def _vel_body(vec_ref, w1_ref, b1_ref, w2_ref, b2_ref, bt_ref,
              velp_ref, sums_ref):
    i = pl.program_id(0)
    cols = []
    w2row = w2_ref[...]
    for ci in range(3):
        sv = _silu(jnp.dot(vec_ref[ci], w1_ref[...],
                           preferred_element_type=_f32) + b1_ref[...])
        r = jnp.sum(sv * w2row, axis=1, keepdims=True) + b2_ref[0, 0]
        cols.append(r)
    rows = i * RB + lax.broadcasted_iota(_i32, (RB, 1), 0)
    ones = jnp.where(rows < NN, 1.0, 0.0)
    velp = jnp.concatenate(cols + [ones, jnp.zeros((RB, 124), _f32)], axis=1)
    velp_ref[...] = velp
    bt = bt_ref[...]
    oh = (lax.broadcasted_iota(_i32, (NGRAPH, 1), 0) == bt).astype(_f32)
    contrib = jnp.dot(oh, velp, preferred_element_type=_f32)

    @pl.when(i == 0)
    def _():
        sums_ref[...] = jnp.zeros((NGRAPH, 128), _f32)

    sums_ref[...] += contrib


def _vel(vec3, vvw1, vvb1, vvw2r, vvb2b, batch):
    return pl.pallas_call(
        _vel_body,
        grid=(NRB,),
        in_specs=[pl.BlockSpec((3, RB, 128), lambda i: (0, i, 0)),
                  pl.BlockSpec((128, 64), lambda i: (0, 0)),
                  pl.BlockSpec((1, 64), lambda i: (0, 0)),
                  pl.BlockSpec((1, 64), lambda i: (0, 0)),
                  pl.BlockSpec((1, 128), lambda i: (0, 0)),
                  pl.BlockSpec((RB,), lambda i: (i,))],
        out_specs=[pl.BlockSpec((RB, 128), lambda i: (i, 0)),
                   pl.BlockSpec((NGRAPH, 128), lambda i: (0, 0))],
        out_shape=[jax.ShapeDtypeStruct((NP, 128), _f32),
                   jax.ShapeDtypeStruct((NGRAPH, 128), _f32)],
    )(vec3, vvw1, vvb1, vvw2r, vvb2b, batch)


def _dec_body(velp_ref, sums_ref, bt_ref, h0_ref, aw1_ref, ab1_ref, aw2_ref,
              ab2_ref, rw1_ref, rb1_ref, rw2_ref, rb2_ref, out_ref):
    i = pl.program_id(0)
    s = sums_ref[...]
    cnt = s[:, 3:4]
    mean = s * (1.0 / jnp.maximum(cnt, 1.0))
    bt = bt_ref[...]
    oh = (lax.broadcasted_iota(_i32, (NGRAPH, 1), 0) == bt).astype(_f32)
    meanrows = lax.dot_general(oh, mean, (((0,), (0,)), ((), ())),
                               preferred_element_type=_f32)
    vel = velp_ref[...] - meanrows
    h = h0_ref[...]
    ha = jnp.dot(_silu(jnp.dot(h, aw1_ref[...], preferred_element_type=_f32)
                       + ab1_ref[...]), aw2_ref[...],
                 preferred_element_type=_f32) + ab2_ref[...]
    hr = jnp.dot(_silu(jnp.dot(h, rw1_ref[...], preferred_element_type=_f32)
                       + rb1_ref[...]), rw2_ref[...],
                 preferred_element_type=_f32) + rb2_ref[...]
    rows = i * RB + lax.broadcasted_iota(_i32, (RB, 1), 0)
    hf = jnp.where(rows < N_AT, ha, hr)
    out_ref[...] = jnp.concatenate(
        [vel[:, 0:3], hf[:, 0:21], jnp.zeros((RB, 8), _f32)], axis=1)


def _decode(velp, sums, batch, h0, aw1, ab1, aw2p, ab2p, rw1, rb1, rw2p, rb2p):
    whole = lambda shape: pl.BlockSpec(shape, lambda i: (0,) * len(shape))
    return pl.pallas_call(
        _dec_body,
        grid=(NRB,),
        in_specs=[pl.BlockSpec((RB, 128), lambda i: (i, 0)),
                  whole((NGRAPH, 128)),
                  pl.BlockSpec((RB,), lambda i: (i,)),
                  pl.BlockSpec((RB, 128), lambda i: (i, 0)),
                  whole((128, 128)), whole((1, 128)), whole((128, 32)),
                  whole((1, 32)),
                  whole((128, 128)), whole((1, 128)), whole((128, 32)),
                  whole((1, 32))],
        out_specs=pl.BlockSpec((RB, 32), lambda i: (i, 0)),
        out_shape=jax.ShapeDtypeStruct((NP, 32), _f32),
    )(velp, sums, batch, h0, aw1, ab1, aw2p, ab2p, rw1, rb1, rw2p, rb2p)


def kernel(xh_atoms, xh_residues, t, mask_atoms, mask_residues,
           edge_index, edge_types, params):
    p = params
    xa = xh_atoms.astype(_f32)
    xr = xh_residues.astype(_f32)

    xh_all = jnp.zeros((NP, 128), _f32)
    xh_all = xh_all.at[:N_AT, 0:ATOM_NF].set(xa[:, 3:])
    xh_all = xh_all.at[N_AT:NN, ATOM_NF:ATOM_NF + RES_NF].set(xr[:, 3:])
    pcat = jnp.concatenate([xa[:, :3], xr[:, :3],
                            jnp.zeros((NP - NN, 3), _f32)], axis=0)
    px = pcat[:, 0]; py = pcat[:, 1]; pz = pcat[:, 2]

    src = jnp.zeros((EP,), _i32).at[:NE].set(edge_index[0].astype(_i32))
    dst = jnp.zeros((EP,), _i32).at[:NE].set(edge_index[1].astype(_i32))
    et2d = (jnp.zeros((EP,), _i32).at[:NE].set(edge_types.astype(_i32))
            .reshape(EP // 128, 128))
    batch = (jnp.full((NP,), -1, _i32)
             .at[:N_AT].set(mask_atoms.astype(_i32))
             .at[N_AT:NN].set(mask_residues.astype(_i32)))

    w1c = (jnp.zeros((128, 128), _f32)
           .at[0:ATOM_NF].set(p['ae_w1'])
           .at[ATOM_NF:ATOM_NF + RES_NF].set(p['re_w1']))
    ab1 = p['ae_b1'].reshape(1, 128); rb1 = p['re_b1'].reshape(1, 128)
    ab2 = p['ae_b2'].reshape(1, 128); rb2 = p['re_b2'].reshape(1, 128)
    win0 = p['win'][:128]
    win1 = p['win'][128:129]
    tb = jnp.broadcast_to(t.reshape(1, 1).astype(_f32), (1, 128))
    ete8 = jnp.zeros((8, 32), _f32).at[0:3].set(p['ete'])
    vvw1 = p['vv_w1']
    vvb1 = p['vv_b1'].reshape(1, 64)
    vvw2r = p['vv_w2'].reshape(1, 64)
    vvb2b = jnp.broadcast_to(p['vv_b2'].reshape(1, 1), (1, 128))
    aw2p = jnp.zeros((128, 32), _f32).at[:, 0:ATOM_NF].set(p['ad_w2'])
    ab2p = jnp.zeros((1, 32), _f32).at[0, 0:ATOM_NF].set(p['ad_b2'])
    rw2p = jnp.zeros((128, 32), _f32).at[:, 0:RES_NF].set(p['rd_w2'])
    rb2p = jnp.zeros((1, 32), _f32).at[0, 0:RES_NF].set(p['rd_b2'])

    h0 = _prologue(xh_all, w1c, ab1, rb1, p['ae_w2'], ab2, p['re_w2'], rb2,
                   win0, win1, tb)
    dxr, dyr, dzr, d2r = _geom(px, py, pz, src, dst)
    ux, uy, uz, filtT = _features(
        dxr.reshape(EP // 128, 128), dyr.reshape(EP // 128, 128),
        dzr.reshape(EP // 128, 128), d2r.reshape(EP // 128, 128),
        et2d, p['mp_wrbf'], ete8)
    ux = ux.reshape(EP); uy = uy.reshape(EP); uz = uz.reshape(EP)

    vec3 = jnp.zeros((3, NP, 128), _f32)
    sd5 = jnp.zeros((8, EP), _i32)
    sd5 = sd5.at[0].set(src).at[1].set(dst)
    sd5 = sd5.at[2].set(lax.bitcast_convert_type(ux, _i32))
    sd5 = sd5.at[3].set(lax.bitcast_convert_type(uy, _i32))
    sd5 = sd5.at[4].set(lax.bitcast_convert_type(uz, _i32))

    for l in range(NLAYERS):
        hw = _hw(h0, p['mp_wh'][l])
        sco = _msg(hw, filtT[l], sd5)
        h0, vec3 = _update(h0, sco.reshape(4, NP, 128), vec3, p['mp_wvec'][l])

    velp, sums = _vel(vec3, vvw1, vvb1, vvw2r, vvb2b, batch)
    res = _decode(velp, sums, batch, h0,
                  p['ad_w1'], p['ad_b1'].reshape(1, 128), aw2p, ab2p,
                  p['rd_w1'], p['rd_b1'].reshape(1, 128), rw2p, rb2p)
    atoms_output = res[:N_AT, 0:3 + ATOM_NF]
    residues_output = res[N_AT:NN, 0:3 + RES_NF]
    return (atoms_output, residues_output)


# speedup vs baseline: 1.7093x; 1.0102x over previous
import functools
import numpy as np
import jax
import jax.numpy as jnp
from jax import lax
from jax.experimental import pallas as pl
from jax.experimental.pallas import tpu as pltpu
from jax.experimental.pallas import tpu_sc as plsc

ATOM_NF = 16; RES_NF = 21; HID = 128; NRBF = 32; CUTOFF = 5.0
NLAYERS = 4; NGRAPH = 32
N_AT = 5000; N_RES = 5000; NN = 10000; NE = 160000
NP = 10240
EP = 163840
NCORE = 2; NSUB = 16
KB = 128
ET = EP // NSUB
NBLK = ET // KB
RPT = NP // NSUB
EPT = EP // (NCORE * NSUB)
RB = 256
NRB = NP // RB

_f32 = jnp.float32
_i32 = jnp.int32

_ALPHA = 5.0 / CUTOFF
_MEANS = np.linspace(float(np.exp(-CUTOFF)), 1.0, NRBF).astype(np.float32)
_BETA = float(((2.0 / NRBF) * (1.0 - np.exp(-CUTOFF))) ** -2)


def _silu(x):
    return x * (1.0 / (1.0 + jnp.exp(-x)))


def _prologue_body(xh_ref, w1_ref, ab1_ref, rb1_ref, w2a_ref, ab2_ref,
                   w2r_ref, rb2_ref, win0_ref, win1_ref, t_ref, h0_ref):
    i = pl.program_id(0)
    x = xh_ref[...]
    rows = i * RB + lax.broadcasted_iota(_i32, (RB, 1), 0)
    is_atom = rows < N_AT
    b1 = jnp.where(is_atom, ab1_ref[...], rb1_ref[...])
    h1 = _silu(jnp.dot(x, w1_ref[...], preferred_element_type=_f32) + b1)
    h2a = jnp.dot(h1, w2a_ref[...], preferred_element_type=_f32) + ab2_ref[...]
    h2r = jnp.dot(h1, w2r_ref[...], preferred_element_type=_f32) + rb2_ref[...]
    h2 = jnp.where(is_atom, h2a, h2r)
    h0 = jnp.dot(h2, win0_ref[...], preferred_element_type=_f32)
    h0_ref[...] = h0 + t_ref[0, 0] * win1_ref[...]


def _prologue(xh_all, w1c, ab1, rb1, w2a, ab2, w2r, rb2, win0, win1, tb):
    whole = lambda shape: pl.BlockSpec(shape, lambda i: (0,) * len(shape))
    return pl.pallas_call(
        _prologue_body,
        grid=(NRB,),
        in_specs=[pl.BlockSpec((RB, 128), lambda i: (i, 0)),
                  whole((128, 128)), whole((1, 128)), whole((1, 128)),
                  whole((128, 128)), whole((1, 128)),
                  whole((128, 128)), whole((1, 128)),
                  whole((128, 128)), whole((1, 128)), whole((1, 128))],
        out_specs=pl.BlockSpec((RB, 128), lambda i: (i, 0)),
        out_shape=jax.ShapeDtypeStruct((NP, 128), _f32),
    )(xh_all, w1c, ab1, rb1, w2a, ab2, w2r, rb2, win0, win1, tb)


def _geom_body(px_hbm, py_hbm, pz_hbm, src_hbm, dst_hbm,
               dx_hbm, dy_hbm, dz_hbm, d2_hbm,
               xs, ys, zs, sbuf, dbuf, ox, oy, oz, o2):
    cid = lax.axis_index("c")
    sid = lax.axis_index("s")
    wid = cid * NSUB + sid
    base = wid * EPT
    pltpu.sync_copy(px_hbm, xs)
    pltpu.sync_copy(py_hbm, ys)
    pltpu.sync_copy(pz_hbm, zs)
    pltpu.sync_copy(src_hbm.at[pl.ds(base, EPT)], sbuf)
    pltpu.sync_copy(dst_hbm.at[pl.ds(base, EPT)], dbuf)

    def body(g, _):
        s16 = sbuf[pl.ds(g * 16, 16)]
        d16 = dbuf[pl.ds(g * 16, 16)]
        dxv = plsc.load_gather(xs, [d16]) - plsc.load_gather(xs, [s16])
        dyv = plsc.load_gather(ys, [d16]) - plsc.load_gather(ys, [s16])
        dzv = plsc.load_gather(zs, [d16]) - plsc.load_gather(zs, [s16])
        d2v = dxv * dxv + dyv * dyv + dzv * dzv
        ox[pl.ds(g * 16, 16)] = dxv
        oy[pl.ds(g * 16, 16)] = dyv
        oz[pl.ds(g * 16, 16)] = dzv
        o2[pl.ds(g * 16, 16)] = d2v
        return 0

    lax.fori_loop(0, EPT // 16, body, 0)
    pltpu.sync_copy(ox, dx_hbm.at[pl.ds(base, EPT)])
    pltpu.sync_copy(oy, dy_hbm.at[pl.ds(base, EPT)])
    pltpu.sync_copy(oz, dz_hbm.at[pl.ds(base, EPT)])
    pltpu.sync_copy(o2, d2_hbm.at[pl.ds(base, EPT)])


def _geom(px, py, pz, src, dst):
    mesh = plsc.VectorSubcoreMesh(core_axis_name="c", subcore_axis_name="s", num_cores=NCORE, num_subcores=NSUB)
    out = jax.ShapeDtypeStruct((EP,), _f32)
    fn = pl.kernel(
        _geom_body,
        out_type=(out, out, out, out),
        mesh=mesh,
        compiler_params=pltpu.CompilerParams(needs_layout_passes=False),
        scratch_types=[pltpu.VMEM((NP,), _f32)] * 3
        + [pltpu.VMEM((EPT,), _i32)] * 2
        + [pltpu.VMEM((EPT,), _f32)] * 4,
    )
    return fn(px, py, pz, src, dst)


def _feat_body(dx_ref, dy_ref, dz_ref, d2_ref, et_ref, wrbf_ref, ete_ref,
               ux_ref, uy_ref, uz_ref, filt_ref):
    i = pl.program_id(0)
    d2 = d2_ref[...]
    dist = jnp.sqrt(d2 + 1e-12)
    inv = 1.0 / (dist + 1e-8)
    ux_ref[...] = dx_ref[...] * inv
    uy_ref[...] = dy_ref[...] * inv
    uz_ref[...] = dz_ref[...] * inv
    ed = jnp.exp(-_ALPHA * dist)
    env = 0.5 * (jnp.cos(jnp.pi * jnp.clip(dist, 0.0, CUTOFF) / CUTOFF) + 1.0)
    m0 = float(np.exp(-CUTOFF))
    means = (m0 + lax.broadcasted_iota(_i32, (NRBF, 1), 0).astype(_f32)
             * ((1.0 - m0) / (NRBF - 1)))
    eidx = (i * 1024 + lax.broadcasted_iota(_i32, (8, 128), 0) * 128
            + lax.broadcasted_iota(_i32, (8, 128), 1))
    valid = eidx < NE
    envm = jnp.where(valid, env, 0.0)
    et = jnp.where(valid, et_ref[...], -1)
    g3 = lax.broadcasted_iota(_i32, (3, 1), 0)
    parts = []
    for r in range(8):
        edr = ed[r:r + 1]
        rbf = jnp.exp(-_BETA * (edr - means) ** 2) * envm[r:r + 1]
        ohf = (g3 == et[r:r + 1]).astype(_f32)
        parts.append(jnp.concatenate(
            [rbf, ohf, jnp.zeros((13, 128), _f32)], axis=0))
    feat = jnp.concatenate(parts, axis=1)
    for l in range(NLAYERS):
        wr = wrbf_ref[l]
        etew = jnp.dot(ete_ref[...], wr, preferred_element_type=_f32)
        w = jnp.concatenate([wr, etew[0:3], jnp.zeros((13, 128), _f32)], axis=0)
        ft = lax.dot_general(w, feat, (((0,), (0,)), ((), ())),
                             preferred_element_type=_f32)
        filt_ref[l] = _silu(ft)


def _features(dxr, dyr, dzr, d2r, et2d, wrbf, ete8):
    row = pl.BlockSpec((8, 128), lambda i: (i, 0))
    outr = jax.ShapeDtypeStruct((EP // 128, 128), _f32)
    return pl.pallas_call(
        _feat_body,
        grid=(EP // 1024,),
        in_specs=[row, row, row, row, row,
                  pl.BlockSpec((NLAYERS, NRBF, 128), lambda i: (0, 0, 0)),
                  pl.BlockSpec((8, 32), lambda i: (0, 0))],
        out_specs=[row, row, row,
                   pl.BlockSpec((NLAYERS, 128, 1024), lambda i: (0, 0, i))],
        out_shape=[outr, outr, outr,
                   jax.ShapeDtypeStruct((NLAYERS, 128, EP), _f32)],
    )(dxr, dyr, dzr, d2r, et2d, wrbf, ete8)


def _hw_body(h0_ref, wh_ref, out_ref):
    out_ref[...] = jnp.dot(h0_ref[...], wh_ref[...],
                           preferred_element_type=_f32)


def _hw(h0, wh):
    return pl.pallas_call(
        _hw_body,
        grid=(NRB,),
        in_specs=[pl.BlockSpec((RB, 128), lambda i: (i, 0)),
                  pl.BlockSpec((128, 128), lambda i: (0, 0))],
        out_specs=pl.BlockSpec((RB, 128), lambda i: (i, 0)),
        out_shape=jax.ShapeDtypeStruct((NP, 128), _f32),
    )(h0, wh)


def _msg_body(hw_hbm, filt_hbm, sd5_hbm, out_hbm,
              acc, pbuf, fbuf, hbufa, hbufb, obuf, dq0, dq1, dq2, dq3,
              ssem, gsem):
    cid = lax.axis_index("c")
    sid = lax.axis_index("s")
    ebase = sid * ET
    dqs = [dq0, dq1, dq2, dq3]

    def zero_hbuf(r, _):
        for q in range(8):
            hbufa[r, pl.ds(q * 16, 16)] = jnp.zeros((16,), _f32)
        return 0

    for p in range(2):
        c = cid * 2 + p
        lax.fori_loop(0, 32, zero_hbuf, 0)

        def zero_acc(q, _):
            pltpu.sync_copy(hbufa, acc.at[pl.ds(sid * RPT + q * 32, 32)])
            return 0
        lax.fori_loop(0, RPT // 32, zero_acc, 0)
        plsc.subcore_barrier()

        e00 = ebase
        pltpu.async_copy(sd5_hbm.at[:, pl.ds(e00, KB)], pbuf.at[0], ssem)
        pltpu.async_copy(filt_hbm.at[pl.ds(c * 32, 32), pl.ds(e00, KB)],
                         fbuf.at[0], ssem)

        def block(b, _):
            sel = lax.rem(b, 2)
            seln = 1 - sel
            pltpu.make_async_copy(
                sd5_hbm.at[:, pl.ds(0, KB)], pbuf.at[sel], ssem).wait()
            pltpu.make_async_copy(
                filt_hbm.at[pl.ds(0, 32), pl.ds(0, KB)],
                fbuf.at[sel], ssem).wait()

            @pl.when(b + 1 < NBLK)
            def _():
                e1 = ebase + (b + 1) * KB
                pltpu.async_copy(sd5_hbm.at[:, pl.ds(e1, KB)],
                                 pbuf.at[seln], ssem)
                pltpu.async_copy(
                    filt_hbm.at[pl.ds(c * 32, 32), pl.ds(e1, KB)],
                    fbuf.at[seln], ssem)

            for qq in range(4):
                for g in range(2):
                    dqs[qq][pl.ds(g * 16, 16)] = (
                        pbuf[sel, 1, pl.ds(qq * 32 + g * 16, 16)])
            gds = [pltpu.async_copy(
                       hw_hbm.at[pbuf.at[sel, 0, pl.ds(0, 32)]],
                       hbufa, gsem),
                   pltpu.async_copy(
                       hw_hbm.at[pbuf.at[sel, 0, pl.ds(32, 32)]],
                       hbufb, gsem)]
            hbase = jnp.full((16,), 32, _i32) * c
            for q in range(4):
                gds[q].wait()
                hb = hbufa if q % 2 == 0 else hbufb
                for g2 in range(2):
                    be = q * 32 + g2 * 16
                    vx = plsc.bitcast(pbuf[sel, 2, pl.ds(be, 16)], _f32)
                    vy = plsc.bitcast(pbuf[sel, 3, pl.ds(be, 16)], _f32)
                    vz = plsc.bitcast(pbuf[sel, 4, pl.ds(be, 16)], _f32)
                    el = lax.iota(_i32, 16) + g2 * 16
                    for h in range(32):
                        f = fbuf[sel, h, pl.ds(be, 16)]
                        w = plsc.load_gather(hb, [el, hbase + h])
                        m = f * w
                        plsc.store_scatter(
                            obuf, [el, jnp.full((16,), h, _i32)], m)
                        plsc.store_scatter(
                            obuf, [el, jnp.full((16,), 32 + h, _i32)],
                            m * vx)
                        plsc.store_scatter(
                            obuf, [el, jnp.full((16,), 64 + h, _i32)],
                            m * vy)
                        plsc.store_scatter(
                            obuf, [el, jnp.full((16,), 96 + h, _i32)],
                            m * vz)
                if q < 2:
                    gds.append(pltpu.async_copy(
                        hw_hbm.at[pbuf.at[sel, 0, pl.ds((q + 2) * 32, 32)]],
                        hbufa if q % 2 == 0 else hbufb, gsem))
                pltpu.sync_copy(obuf, acc.at[dqs[q]], add=True)
            return 0

        lax.fori_loop(0, NBLK, block, 0)
        plsc.subcore_barrier()
        pltpu.sync_copy(acc.at[pl.ds(sid * RPT, RPT)],
                        out_hbm.at[pl.ds(c * NP + sid * RPT, RPT)])
        plsc.subcore_barrier()


def _msg(hw, filt_l, sd5):
    mesh = plsc.VectorSubcoreMesh(core_axis_name="c", subcore_axis_name="s", num_cores=NCORE, num_subcores=NSUB)
    fn = pl.kernel(
        _msg_body,
        out_type=jax.ShapeDtypeStruct((4 * NP, 128), _f32),
        mesh=mesh,
        compiler_params=pltpu.CompilerParams(needs_layout_passes=False),
        scratch_types=[
            pltpu.VMEM_SHARED((NP, 128), _f32),
            pltpu.VMEM((2, 8, KB), _i32),
            pltpu.VMEM((2, 32, KB), _f32),
            pltpu.VMEM((32, 128), _f32),
            pltpu.VMEM((32, 128), _f32),
            pltpu.VMEM((32, 128), _f32),
            pltpu.VMEM((32,), _i32),
            pltpu.VMEM((32,), _i32),
            pltpu.VMEM((32,), _i32),
            pltpu.VMEM((32,), _i32),
            pltpu.SemaphoreType.DMA,
            pltpu.SemaphoreType.DMA,
        ],
    )
    return fn(hw, filt_l, sd5)


def _update_body(h0_ref, sco_ref, vec_ref, wv_ref, h0o_ref, veco_ref):
    s = sco_ref[...]
    agg = jnp.concatenate([s[k, :, 0:32] for k in range(4)], axis=1)
    h0o_ref[...] = h0_ref[...] + _silu(agg)
    wv = wv_ref[...]
    for ci in range(3):
        c0 = 32 * (ci + 1)
        d = jnp.concatenate([s[k, :, c0:c0 + 32] for k in range(4)], axis=1)
        veco_ref[ci] = jnp.dot(vec_ref[ci] + d, wv, preferred_element_type=_f32)


def _update(h0, sco4, vec3, wv):
    return pl.pallas_call(
        _update_body,
        grid=(NRB,),
        in_specs=[pl.BlockSpec((RB, 128), lambda i: (i, 0)),
                  pl.BlockSpec((4, RB, 128), lambda i: (0, i, 0)),
                  pl.BlockSpec((3, RB, 128), lambda i: (0, i, 0)),
                  pl.BlockSpec((128, 128), lambda i: (0, 0))],
        out_specs=[pl.BlockSpec((RB, 128), lambda i: (i, 0)),
                   pl.BlockSpec((3, RB, 128), lambda i: (0, i, 0))],
        out_shape=[jax.ShapeDtypeStruct((NP, 128), _f32),
                   jax.ShapeDtypeStruct((3, NP, 128), _f32)],
    )(h0, sco4, vec3, wv)


def _vel_body(vec_ref, w1_ref, b1_ref, w2_ref, b2_ref, bt_ref,
              velp_ref, sums_ref):
    i = pl.program_id(0)
    cols = []
    w2row = w2_ref[...]
    for ci in range(3):
        sv = _silu(jnp.dot(vec_ref[ci], w1_ref[...],
                           preferred_element_type=_f32) + b1_ref[...])
        r = jnp.sum(sv * w2row, axis=1, keepdims=True) + b2_ref[0, 0]
        cols.append(r)
    rows = i * RB + lax.broadcasted_iota(_i32, (RB, 1), 0)
    ones = jnp.where(rows < NN, 1.0, 0.0)
    velp = jnp.concatenate(cols + [ones, jnp.zeros((RB, 124), _f32)], axis=1)
    velp_ref[...] = velp
    bt = bt_ref[...]
    oh = (lax.broadcasted_iota(_i32, (NGRAPH, 1), 0) == bt).astype(_f32)
    contrib = jnp.dot(oh, velp, preferred_element_type=_f32)

    @pl.when(i == 0)
    def _():
        sums_ref[...] = jnp.zeros((NGRAPH, 128), _f32)

    sums_ref[...] += contrib


def _vel(vec3, vvw1, vvb1, vvw2r, vvb2b, batch):
    return pl.pallas_call(
        _vel_body,
        grid=(NRB,),
        in_specs=[pl.BlockSpec((3, RB, 128), lambda i: (0, i, 0)),
                  pl.BlockSpec((128, 64), lambda i: (0, 0)),
                  pl.BlockSpec((1, 64), lambda i: (0, 0)),
                  pl.BlockSpec((1, 64), lambda i: (0, 0)),
                  pl.BlockSpec((1, 128), lambda i: (0, 0)),
                  pl.BlockSpec((RB,), lambda i: (i,))],
        out_specs=[pl.BlockSpec((RB, 128), lambda i: (i, 0)),
                   pl.BlockSpec((NGRAPH, 128), lambda i: (0, 0))],
        out_shape=[jax.ShapeDtypeStruct((NP, 128), _f32),
                   jax.ShapeDtypeStruct((NGRAPH, 128), _f32)],
    )(vec3, vvw1, vvb1, vvw2r, vvb2b, batch)


def _dec_body(velp_ref, sums_ref, bt_ref, h0_ref, aw1_ref, ab1_ref, aw2_ref,
              ab2_ref, rw1_ref, rb1_ref, rw2_ref, rb2_ref, out_ref):
    i = pl.program_id(0)
    s = sums_ref[...]
    cnt = s[:, 3:4]
    mean = s * (1.0 / jnp.maximum(cnt, 1.0))
    bt = bt_ref[...]
    oh = (lax.broadcasted_iota(_i32, (NGRAPH, 1), 0) == bt).astype(_f32)
    meanrows = lax.dot_general(oh, mean, (((0,), (0,)), ((), ())),
                               preferred_element_type=_f32)
    vel = velp_ref[...] - meanrows
    h = h0_ref[...]
    ha = jnp.dot(_silu(jnp.dot(h, aw1_ref[...], preferred_element_type=_f32)
                       + ab1_ref[...]), aw2_ref[...],
                 preferred_element_type=_f32) + ab2_ref[...]
    hr = jnp.dot(_silu(jnp.dot(h, rw1_ref[...], preferred_element_type=_f32)
                       + rb1_ref[...]), rw2_ref[...],
                 preferred_element_type=_f32) + rb2_ref[...]
    rows = i * RB + lax.broadcasted_iota(_i32, (RB, 1), 0)
    hf = jnp.where(rows < N_AT, ha, hr)
    out_ref[...] = jnp.concatenate(
        [vel[:, 0:3], hf[:, 0:21], jnp.zeros((RB, 8), _f32)], axis=1)


def _decode(velp, sums, batch, h0, aw1, ab1, aw2p, ab2p, rw1, rb1, rw2p, rb2p):
    whole = lambda shape: pl.BlockSpec(shape, lambda i: (0,) * len(shape))
    return pl.pallas_call(
        _dec_body,
        grid=(NRB,),
        in_specs=[pl.BlockSpec((RB, 128), lambda i: (i, 0)),
                  whole((NGRAPH, 128)),
                  pl.BlockSpec((RB,), lambda i: (i,)),
                  pl.BlockSpec((RB, 128), lambda i: (i, 0)),
                  whole((128, 128)), whole((1, 128)), whole((128, 32)),
                  whole((1, 32)),
                  whole((128, 128)), whole((1, 128)), whole((128, 32)),
                  whole((1, 32))],
        out_specs=pl.BlockSpec((RB, 32), lambda i: (i, 0)),
        out_shape=jax.ShapeDtypeStruct((NP, 32), _f32),
    )(velp, sums, batch, h0, aw1, ab1, aw2p, ab2p, rw1, rb1, rw2p, rb2p)


def kernel(xh_atoms, xh_residues, t, mask_atoms, mask_residues,
           edge_index, edge_types, params):
    p = params
    xa = xh_atoms.astype(_f32)
    xr = xh_residues.astype(_f32)

    xh_all = jnp.zeros((NP, 128), _f32)
    xh_all = xh_all.at[:N_AT, 0:ATOM_NF].set(xa[:, 3:])
    xh_all = xh_all.at[N_AT:NN, ATOM_NF:ATOM_NF + RES_NF].set(xr[:, 3:])
    pcat = jnp.concatenate([xa[:, :3], xr[:, :3],
                            jnp.zeros((NP - NN, 3), _f32)], axis=0)
    px = pcat[:, 0]; py = pcat[:, 1]; pz = pcat[:, 2]

    src = jnp.zeros((EP,), _i32).at[:NE].set(edge_index[0].astype(_i32))
    dst = jnp.zeros((EP,), _i32).at[:NE].set(edge_index[1].astype(_i32))
    et2d = (jnp.zeros((EP,), _i32).at[:NE].set(edge_types.astype(_i32))
            .reshape(EP // 128, 128))
    batch = (jnp.full((NP,), -1, _i32)
             .at[:N_AT].set(mask_atoms.astype(_i32))
             .at[N_AT:NN].set(mask_residues.astype(_i32)))

    w1c = (jnp.zeros((128, 128), _f32)
           .at[0:ATOM_NF].set(p['ae_w1'])
           .at[ATOM_NF:ATOM_NF + RES_NF].set(p['re_w1']))
    ab1 = p['ae_b1'].reshape(1, 128); rb1 = p['re_b1'].reshape(1, 128)
    ab2 = p['ae_b2'].reshape(1, 128); rb2 = p['re_b2'].reshape(1, 128)
    win0 = p['win'][:128]
    win1 = p['win'][128:129]
    tb = jnp.broadcast_to(t.reshape(1, 1).astype(_f32), (1, 128))
    ete8 = jnp.zeros((8, 32), _f32).at[0:3].set(p['ete'])
    vvw1 = p['vv_w1']
    vvb1 = p['vv_b1'].reshape(1, 64)
    vvw2r = p['vv_w2'].reshape(1, 64)
    vvb2b = jnp.broadcast_to(p['vv_b2'].reshape(1, 1), (1, 128))
    aw2p = jnp.zeros((128, 32), _f32).at[:, 0:ATOM_NF].set(p['ad_w2'])
    ab2p = jnp.zeros((1, 32), _f32).at[0, 0:ATOM_NF].set(p['ad_b2'])
    rw2p = jnp.zeros((128, 32), _f32).at[:, 0:RES_NF].set(p['rd_w2'])
    rb2p = jnp.zeros((1, 32), _f32).at[0, 0:RES_NF].set(p['rd_b2'])

    h0 = _prologue(xh_all, w1c, ab1, rb1, p['ae_w2'], ab2, p['re_w2'], rb2,
                   win0, win1, tb)
    dxr, dyr, dzr, d2r = _geom(px, py, pz, src, dst)
    ux, uy, uz, filtT = _features(
        dxr.reshape(EP // 128, 128), dyr.reshape(EP // 128, 128),
        dzr.reshape(EP // 128, 128), d2r.reshape(EP // 128, 128),
        et2d, p['mp_wrbf'], ete8)
    ux = ux.reshape(EP); uy = uy.reshape(EP); uz = uz.reshape(EP)

    vec3 = jnp.zeros((3, NP, 128), _f32)
    sd5 = jnp.zeros((8, EP), _i32)
    sd5 = sd5.at[0].set(src).at[1].set(dst)
    sd5 = sd5.at[2].set(lax.bitcast_convert_type(ux, _i32))
    sd5 = sd5.at[3].set(lax.bitcast_convert_type(uy, _i32))
    sd5 = sd5.at[4].set(lax.bitcast_convert_type(uz, _i32))

    for l in range(NLAYERS):
        hw = _hw(h0, p['mp_wh'][l])
        sco = _msg(hw, filtT[l], sd5)
        h0, vec3 = _update(h0, sco.reshape(4, NP, 128), vec3, p['mp_wvec'][l])

    velp, sums = _vel(vec3, vvw1, vvb1, vvw2r, vvb2b, batch)
    res = _decode(velp, sums, batch, h0,
                  p['ad_w1'], p['ad_b1'].reshape(1, 128), aw2p, ab2p,
                  p['rd_w1'], p['rd_b1'].reshape(1, 128), rw2p, rb2p)
    atoms_output = res[:N_AT, 0:3 + ATOM_NF]
    residues_output = res[N_AT:NN, 0:3 + RES_NF]
    return (atoms_output, residues_output)


# speedup vs baseline: 1.8309x; 1.0712x over previous
import functools
import numpy as np
import jax
import jax.numpy as jnp
from jax import lax
from jax.experimental import pallas as pl
from jax.experimental.pallas import tpu as pltpu
from jax.experimental.pallas import tpu_sc as plsc

ATOM_NF = 16; RES_NF = 21; HID = 128; NRBF = 32; CUTOFF = 5.0
NLAYERS = 4; NGRAPH = 32
N_AT = 5000; N_RES = 5000; NN = 10000; NE = 160000
NP = 10240
EP = 163840
NCORE = 2; NSUB = 16
KB = 128
ET = EP // NSUB
NBLK = ET // KB
RPT = NP // NSUB
EPT = EP // (NCORE * NSUB)
RB = 256
NRB = NP // RB

_f32 = jnp.float32
_i32 = jnp.int32

_ALPHA = 5.0 / CUTOFF
_MEANS = np.linspace(float(np.exp(-CUTOFF)), 1.0, NRBF).astype(np.float32)
_BETA = float(((2.0 / NRBF) * (1.0 - np.exp(-CUTOFF))) ** -2)


def _silu(x):
    return x * (1.0 / (1.0 + jnp.exp(-x)))


def _prologue_body(xh_ref, w1_ref, ab1_ref, rb1_ref, w2a_ref, ab2_ref,
                   w2r_ref, rb2_ref, win0_ref, win1_ref, t_ref, h0_ref):
    i = pl.program_id(0)
    x = xh_ref[...]
    rows = i * RB + lax.broadcasted_iota(_i32, (RB, 1), 0)
    is_atom = rows < N_AT
    b1 = jnp.where(is_atom, ab1_ref[...], rb1_ref[...])
    h1 = _silu(jnp.dot(x, w1_ref[...], preferred_element_type=_f32) + b1)
    h2a = jnp.dot(h1, w2a_ref[...], preferred_element_type=_f32) + ab2_ref[...]
    h2r = jnp.dot(h1, w2r_ref[...], preferred_element_type=_f32) + rb2_ref[...]
    h2 = jnp.where(is_atom, h2a, h2r)
    h0 = jnp.dot(h2, win0_ref[...], preferred_element_type=_f32)
    h0_ref[...] = h0 + t_ref[0, 0] * win1_ref[...]


def _prologue(xh_all, w1c, ab1, rb1, w2a, ab2, w2r, rb2, win0, win1, tb):
    whole = lambda shape: pl.BlockSpec(shape, lambda i: (0,) * len(shape))
    return pl.pallas_call(
        _prologue_body,
        grid=(NRB,),
        in_specs=[pl.BlockSpec((RB, 128), lambda i: (i, 0)),
                  whole((128, 128)), whole((1, 128)), whole((1, 128)),
                  whole((128, 128)), whole((1, 128)),
                  whole((128, 128)), whole((1, 128)),
                  whole((128, 128)), whole((1, 128)), whole((1, 128))],
        out_specs=pl.BlockSpec((RB, 128), lambda i: (i, 0)),
        out_shape=jax.ShapeDtypeStruct((NP, 128), _f32),
    )(xh_all, w1c, ab1, rb1, w2a, ab2, w2r, rb2, win0, win1, tb)


def _geom_body(px_hbm, py_hbm, pz_hbm, src_hbm, dst_hbm,
               dx_hbm, dy_hbm, dz_hbm, d2_hbm,
               xs, ys, zs, sbuf, dbuf, ox, oy, oz, o2):
    cid = lax.axis_index("c")
    sid = lax.axis_index("s")
    wid = cid * NSUB + sid
    base = wid * EPT
    pltpu.sync_copy(px_hbm, xs)
    pltpu.sync_copy(py_hbm, ys)
    pltpu.sync_copy(pz_hbm, zs)
    pltpu.sync_copy(src_hbm.at[pl.ds(base, EPT)], sbuf)
    pltpu.sync_copy(dst_hbm.at[pl.ds(base, EPT)], dbuf)

    def body(g, _):
        s16 = sbuf[pl.ds(g * 16, 16)]
        d16 = dbuf[pl.ds(g * 16, 16)]
        dxv = plsc.load_gather(xs, [d16]) - plsc.load_gather(xs, [s16])
        dyv = plsc.load_gather(ys, [d16]) - plsc.load_gather(ys, [s16])
        dzv = plsc.load_gather(zs, [d16]) - plsc.load_gather(zs, [s16])
        d2v = dxv * dxv + dyv * dyv + dzv * dzv
        ox[pl.ds(g * 16, 16)] = dxv
        oy[pl.ds(g * 16, 16)] = dyv
        oz[pl.ds(g * 16, 16)] = dzv
        o2[pl.ds(g * 16, 16)] = d2v
        return 0

    lax.fori_loop(0, EPT // 16, body, 0)
    pltpu.sync_copy(ox, dx_hbm.at[pl.ds(base, EPT)])
    pltpu.sync_copy(oy, dy_hbm.at[pl.ds(base, EPT)])
    pltpu.sync_copy(oz, dz_hbm.at[pl.ds(base, EPT)])
    pltpu.sync_copy(o2, d2_hbm.at[pl.ds(base, EPT)])


def _geom(px, py, pz, src, dst):
    mesh = plsc.VectorSubcoreMesh(core_axis_name="c", subcore_axis_name="s", num_cores=NCORE, num_subcores=NSUB)
    out = jax.ShapeDtypeStruct((EP,), _f32)
    fn = pl.kernel(
        _geom_body,
        out_type=(out, out, out, out),
        mesh=mesh,
        compiler_params=pltpu.CompilerParams(needs_layout_passes=False),
        scratch_types=[pltpu.VMEM((NP,), _f32)] * 3
        + [pltpu.VMEM((EPT,), _i32)] * 2
        + [pltpu.VMEM((EPT,), _f32)] * 4,
    )
    return fn(px, py, pz, src, dst)


def _feat_body(dx_ref, dy_ref, dz_ref, d2_ref, et_ref, wrbf_ref, ete_ref,
               ux_ref, uy_ref, uz_ref, filt_ref):
    i = pl.program_id(0)
    d2 = d2_ref[...]
    dist = jnp.sqrt(d2 + 1e-12)
    inv = 1.0 / (dist + 1e-8)
    ux_ref[...] = dx_ref[...] * inv
    uy_ref[...] = dy_ref[...] * inv
    uz_ref[...] = dz_ref[...] * inv
    ed = jnp.exp(-_ALPHA * dist)
    env = 0.5 * (jnp.cos(jnp.pi * jnp.clip(dist, 0.0, CUTOFF) / CUTOFF) + 1.0)
    m0 = float(np.exp(-CUTOFF))
    means = (m0 + lax.broadcasted_iota(_i32, (NRBF, 1), 0).astype(_f32)
             * ((1.0 - m0) / (NRBF - 1)))
    eidx = (i * 1024 + lax.broadcasted_iota(_i32, (8, 128), 0) * 128
            + lax.broadcasted_iota(_i32, (8, 128), 1))
    valid = eidx < NE
    envm = jnp.where(valid, env, 0.0)
    et = jnp.where(valid, et_ref[...], -1)
    g3 = lax.broadcasted_iota(_i32, (3, 1), 0)
    parts = []
    for r in range(8):
        edr = ed[r:r + 1]
        rbf = jnp.exp(-_BETA * (edr - means) ** 2) * envm[r:r + 1]
        ohf = (g3 == et[r:r + 1]).astype(_f32)
        parts.append(jnp.concatenate(
            [rbf, ohf, jnp.zeros((13, 128), _f32)], axis=0))
    feat = jnp.concatenate(parts, axis=1)
    for l in range(NLAYERS):
        wr = wrbf_ref[l]
        etew = jnp.dot(ete_ref[...], wr, preferred_element_type=_f32)
        w = jnp.concatenate([wr, etew[0:3], jnp.zeros((13, 128), _f32)], axis=0)
        ft = lax.dot_general(w, feat, (((0,), (0,)), ((), ())),
                             preferred_element_type=_f32)
        filt_ref[l] = _silu(ft)


def _features(dxr, dyr, dzr, d2r, et2d, wrbf, ete8):
    row = pl.BlockSpec((8, 128), lambda i: (i, 0))
    outr = jax.ShapeDtypeStruct((EP // 128, 128), _f32)
    return pl.pallas_call(
        _feat_body,
        grid=(EP // 1024,),
        in_specs=[row, row, row, row, row,
                  pl.BlockSpec((NLAYERS, NRBF, 128), lambda i: (0, 0, 0)),
                  pl.BlockSpec((8, 32), lambda i: (0, 0))],
        out_specs=[row, row, row,
                   pl.BlockSpec((NLAYERS, 128, 1024), lambda i: (0, 0, i))],
        out_shape=[outr, outr, outr,
                   jax.ShapeDtypeStruct((NLAYERS, 128, EP), _f32)],
    )(dxr, dyr, dzr, d2r, et2d, wrbf, ete8)


def _hw_body(h0_ref, wh_ref, out_ref):
    out_ref[...] = jnp.dot(h0_ref[...], wh_ref[...],
                           preferred_element_type=_f32)


def _hw(h0, wh):
    return pl.pallas_call(
        _hw_body,
        grid=(NRB,),
        in_specs=[pl.BlockSpec((RB, 128), lambda i: (i, 0)),
                  pl.BlockSpec((128, 128), lambda i: (0, 0))],
        out_specs=pl.BlockSpec((RB, 128), lambda i: (i, 0)),
        out_shape=jax.ShapeDtypeStruct((NP, 128), _f32),
    )(h0, wh)


def _msg_body(hw_hbm, filt_hbm, sd5_hbm, out_hbm,
              acc, pbuf, fbuf, hbufa, hbufb, obufa, obufb,
              dq0, dq1, dq2, dq3, dq4, dq5, dq6, dq7,
              ssem, gsem, osem):
    cid = lax.axis_index("c")
    sid = lax.axis_index("s")
    ebase = sid * ET
    dqs = [dq0, dq1, dq2, dq3, dq4, dq5, dq6, dq7]

    def zero_obuf(r, _):
        for q in range(8):
            obufa[r, pl.ds(q * 16, 16)] = jnp.zeros((16,), _f32)
        return 0

    def drain_scatter(slot):
        ob = obufa if slot == 0 else obufb
        pltpu.make_async_copy(ob, acc.at[dqs[slot]], osem).wait()

    for p in range(2):
        c = cid * 2 + p
        lax.fori_loop(0, 16, zero_obuf, 0)

        def zero_acc(q, _):
            pltpu.sync_copy(obufa, acc.at[pl.ds(sid * RPT + q * 16, 16)])
            return 0
        lax.fori_loop(0, RPT // 16, zero_acc, 0)
        plsc.subcore_barrier()

        pltpu.async_copy(sd5_hbm.at[:, pl.ds(ebase, KB)], pbuf.at[0], ssem)
        pltpu.async_copy(filt_hbm.at[pl.ds(c * 32, 32), pl.ds(ebase, KB)],
                         fbuf.at[0], ssem)

        def block(b, _):
            sel = lax.rem(b, 2)
            seln = 1 - sel
            pltpu.make_async_copy(
                sd5_hbm.at[:, pl.ds(0, KB)], pbuf.at[sel], ssem).wait()
            pltpu.make_async_copy(
                filt_hbm.at[pl.ds(0, 32), pl.ds(0, KB)],
                fbuf.at[sel], ssem).wait()

            @pl.when(b + 1 < NBLK)
            def _():
                e1 = ebase + (b + 1) * KB
                pltpu.async_copy(sd5_hbm.at[:, pl.ds(e1, KB)],
                                 pbuf.at[seln], ssem)
                pltpu.async_copy(
                    filt_hbm.at[pl.ds(c * 32, 32), pl.ds(e1, KB)],
                    fbuf.at[seln], ssem)

            @pl.when(b > 0)
            def _():
                drain_scatter(0)
                drain_scatter(1)

            for o in range(8):
                dqs[o][...] = pbuf[sel, 1, pl.ds(o * 16, 16)]
            gds = [pltpu.async_copy(
                       hw_hbm.at[pbuf.at[sel, 0, pl.ds(0, 16)]],
                       hbufa, gsem),
                   pltpu.async_copy(
                       hw_hbm.at[pbuf.at[sel, 0, pl.ds(16, 16)]],
                       hbufb, gsem)]
            hbase = jnp.full((16,), 32, _i32) * c
            el = lax.iota(_i32, 16)
            for o in range(8):
                gds[o].wait()
                hb = hbufa if o % 2 == 0 else hbufb
                ob = obufa if o % 2 == 0 else obufb
                if o >= 2:
                    drain_scatter(o % 2)
                be = o * 16
                vx = plsc.bitcast(pbuf[sel, 2, pl.ds(be, 16)], _f32)
                vy = plsc.bitcast(pbuf[sel, 3, pl.ds(be, 16)], _f32)
                vz = plsc.bitcast(pbuf[sel, 4, pl.ds(be, 16)], _f32)
                for h in range(32):
                    f = fbuf[sel, h, pl.ds(be, 16)]
                    w = plsc.load_gather(hb, [el, hbase + h])
                    m = f * w
                    plsc.store_scatter(
                        ob, [el, jnp.full((16,), h, _i32)], m)
                    plsc.store_scatter(
                        ob, [el, jnp.full((16,), 32 + h, _i32)], m * vx)
                    plsc.store_scatter(
                        ob, [el, jnp.full((16,), 64 + h, _i32)], m * vy)
                    plsc.store_scatter(
                        ob, [el, jnp.full((16,), 96 + h, _i32)], m * vz)
                if o < 6:
                    gds.append(pltpu.async_copy(
                        hw_hbm.at[pbuf.at[sel, 0, pl.ds((o + 2) * 16, 16)]],
                        hbufa if o % 2 == 0 else hbufb, gsem))
                pltpu.async_copy(ob, acc.at[dqs[o]], osem, add=True)
            return 0

        lax.fori_loop(0, NBLK, block, 0)
        drain_scatter(0)
        drain_scatter(1)
        plsc.subcore_barrier()
        pltpu.sync_copy(acc.at[pl.ds(sid * RPT, RPT)],
                        out_hbm.at[pl.ds(c * NP + sid * RPT, RPT)])
        plsc.subcore_barrier()


def _msg(hw, filt_l, sd5):
    mesh = plsc.VectorSubcoreMesh(core_axis_name="c", subcore_axis_name="s", num_cores=NCORE, num_subcores=NSUB)
    fn = pl.kernel(
        _msg_body,
        out_type=jax.ShapeDtypeStruct((4 * NP, 128), _f32),
        mesh=mesh,
        compiler_params=pltpu.CompilerParams(needs_layout_passes=False),
        scratch_types=[
            pltpu.VMEM_SHARED((NP, 128), _f32),
            pltpu.VMEM((2, 8, KB), _i32),
            pltpu.VMEM((2, 32, KB), _f32),
            pltpu.VMEM((16, 128), _f32),
            pltpu.VMEM((16, 128), _f32),
            pltpu.VMEM((16, 128), _f32),
            pltpu.VMEM((16, 128), _f32),
            pltpu.VMEM((16,), _i32),
            pltpu.VMEM((16,), _i32),
            pltpu.VMEM((16,), _i32),
            pltpu.VMEM((16,), _i32),
            pltpu.VMEM((16,), _i32),
            pltpu.VMEM((16,), _i32),
            pltpu.VMEM((16,), _i32),
            pltpu.VMEM((16,), _i32),
            pltpu.SemaphoreType.DMA,
            pltpu.SemaphoreType.DMA,
            pltpu.SemaphoreType.DMA,
        ],
    )
    return fn(hw, filt_l, sd5)


def _update_body(h0_ref, sco_ref, vec_ref, wv_ref, h0o_ref, veco_ref):
    s = sco_ref[...]
    agg = jnp.concatenate([s[k, :, 0:32] for k in range(4)], axis=1)
    h0o_ref[...] = h0_ref[...] + _silu(agg)
    wv = wv_ref[...]
    for ci in range(3):
        c0 = 32 * (ci + 1)
        d = jnp.concatenate([s[k, :, c0:c0 + 32] for k in range(4)], axis=1)
        veco_ref[ci] = jnp.dot(vec_ref[ci] + d, wv, preferred_element_type=_f32)


def _update(h0, sco4, vec3, wv):
    return pl.pallas_call(
        _update_body,
        grid=(NRB,),
        in_specs=[pl.BlockSpec((RB, 128), lambda i: (i, 0)),
                  pl.BlockSpec((4, RB, 128), lambda i: (0, i, 0)),
                  pl.BlockSpec((3, RB, 128), lambda i: (0, i, 0)),
                  pl.BlockSpec((128, 128), lambda i: (0, 0))],
        out_specs=[pl.BlockSpec((RB, 128), lambda i: (i, 0)),
                   pl.BlockSpec((3, RB, 128), lambda i: (0, i, 0))],
        out_shape=[jax.ShapeDtypeStruct((NP, 128), _f32),
                   jax.ShapeDtypeStruct((3, NP, 128), _f32)],
    )(h0, sco4, vec3, wv)


def _vel_body(vec_ref, w1_ref, b1_ref, w2_ref, b2_ref, bt_ref,
              velp_ref, sums_ref):
    i = pl.program_id(0)
    cols = []
    w2row = w2_ref[...]
    for ci in range(3):
        sv = _silu(jnp.dot(vec_ref[ci], w1_ref[...],
                           preferred_element_type=_f32) + b1_ref[...])
        r = jnp.sum(sv * w2row, axis=1, keepdims=True) + b2_ref[0, 0]
        cols.append(r)
    rows = i * RB + lax.broadcasted_iota(_i32, (RB, 1), 0)
    ones = jnp.where(rows < NN, 1.0, 0.0)
    velp = jnp.concatenate(cols + [ones, jnp.zeros((RB, 124), _f32)], axis=1)
    velp_ref[...] = velp
    bt = bt_ref[...]
    oh = (lax.broadcasted_iota(_i32, (NGRAPH, 1), 0) == bt).astype(_f32)
    contrib = jnp.dot(oh, velp, preferred_element_type=_f32)

    @pl.when(i == 0)
    def _():
        sums_ref[...] = jnp.zeros((NGRAPH, 128), _f32)

    sums_ref[...] += contrib


def _vel(vec3, vvw1, vvb1, vvw2r, vvb2b, batch):
    return pl.pallas_call(
        _vel_body,
        grid=(NRB,),
        in_specs=[pl.BlockSpec((3, RB, 128), lambda i: (0, i, 0)),
                  pl.BlockSpec((128, 64), lambda i: (0, 0)),
                  pl.BlockSpec((1, 64), lambda i: (0, 0)),
                  pl.BlockSpec((1, 64), lambda i: (0, 0)),
                  pl.BlockSpec((1, 128), lambda i: (0, 0)),
                  pl.BlockSpec((RB,), lambda i: (i,))],
        out_specs=[pl.BlockSpec((RB, 128), lambda i: (i, 0)),
                   pl.BlockSpec((NGRAPH, 128), lambda i: (0, 0))],
        out_shape=[jax.ShapeDtypeStruct((NP, 128), _f32),
                   jax.ShapeDtypeStruct((NGRAPH, 128), _f32)],
    )(vec3, vvw1, vvb1, vvw2r, vvb2b, batch)


def _dec_body(velp_ref, sums_ref, bt_ref, h0_ref, aw1_ref, ab1_ref, aw2_ref,
              ab2_ref, rw1_ref, rb1_ref, rw2_ref, rb2_ref, out_ref):
    i = pl.program_id(0)
    s = sums_ref[...]
    cnt = s[:, 3:4]
    mean = s * (1.0 / jnp.maximum(cnt, 1.0))
    bt = bt_ref[...]
    oh = (lax.broadcasted_iota(_i32, (NGRAPH, 1), 0) == bt).astype(_f32)
    meanrows = lax.dot_general(oh, mean, (((0,), (0,)), ((), ())),
                               preferred_element_type=_f32)
    vel = velp_ref[...] - meanrows
    h = h0_ref[...]
    ha = jnp.dot(_silu(jnp.dot(h, aw1_ref[...], preferred_element_type=_f32)
                       + ab1_ref[...]), aw2_ref[...],
                 preferred_element_type=_f32) + ab2_ref[...]
    hr = jnp.dot(_silu(jnp.dot(h, rw1_ref[...], preferred_element_type=_f32)
                       + rb1_ref[...]), rw2_ref[...],
                 preferred_element_type=_f32) + rb2_ref[...]
    rows = i * RB + lax.broadcasted_iota(_i32, (RB, 1), 0)
    hf = jnp.where(rows < N_AT, ha, hr)
    out_ref[...] = jnp.concatenate(
        [vel[:, 0:3], hf[:, 0:21], jnp.zeros((RB, 8), _f32)], axis=1)


def _decode(velp, sums, batch, h0, aw1, ab1, aw2p, ab2p, rw1, rb1, rw2p, rb2p):
    whole = lambda shape: pl.BlockSpec(shape, lambda i: (0,) * len(shape))
    return pl.pallas_call(
        _dec_body,
        grid=(NRB,),
        in_specs=[pl.BlockSpec((RB, 128), lambda i: (i, 0)),
                  whole((NGRAPH, 128)),
                  pl.BlockSpec((RB,), lambda i: (i,)),
                  pl.BlockSpec((RB, 128), lambda i: (i, 0)),
                  whole((128, 128)), whole((1, 128)), whole((128, 32)),
                  whole((1, 32)),
                  whole((128, 128)), whole((1, 128)), whole((128, 32)),
                  whole((1, 32))],
        out_specs=pl.BlockSpec((RB, 32), lambda i: (i, 0)),
        out_shape=jax.ShapeDtypeStruct((NP, 32), _f32),
    )(velp, sums, batch, h0, aw1, ab1, aw2p, ab2p, rw1, rb1, rw2p, rb2p)


def kernel(xh_atoms, xh_residues, t, mask_atoms, mask_residues,
           edge_index, edge_types, params):
    p = params
    xa = xh_atoms.astype(_f32)
    xr = xh_residues.astype(_f32)

    xh_all = jnp.zeros((NP, 128), _f32)
    xh_all = xh_all.at[:N_AT, 0:ATOM_NF].set(xa[:, 3:])
    xh_all = xh_all.at[N_AT:NN, ATOM_NF:ATOM_NF + RES_NF].set(xr[:, 3:])
    pcat = jnp.concatenate([xa[:, :3], xr[:, :3],
                            jnp.zeros((NP - NN, 3), _f32)], axis=0)
    px = pcat[:, 0]; py = pcat[:, 1]; pz = pcat[:, 2]

    src = jnp.zeros((EP,), _i32).at[:NE].set(edge_index[0].astype(_i32))
    dst = jnp.zeros((EP,), _i32).at[:NE].set(edge_index[1].astype(_i32))
    et2d = (jnp.zeros((EP,), _i32).at[:NE].set(edge_types.astype(_i32))
            .reshape(EP // 128, 128))
    batch = (jnp.full((NP,), -1, _i32)
             .at[:N_AT].set(mask_atoms.astype(_i32))
             .at[N_AT:NN].set(mask_residues.astype(_i32)))

    w1c = (jnp.zeros((128, 128), _f32)
           .at[0:ATOM_NF].set(p['ae_w1'])
           .at[ATOM_NF:ATOM_NF + RES_NF].set(p['re_w1']))
    ab1 = p['ae_b1'].reshape(1, 128); rb1 = p['re_b1'].reshape(1, 128)
    ab2 = p['ae_b2'].reshape(1, 128); rb2 = p['re_b2'].reshape(1, 128)
    win0 = p['win'][:128]
    win1 = p['win'][128:129]
    tb = jnp.broadcast_to(t.reshape(1, 1).astype(_f32), (1, 128))
    ete8 = jnp.zeros((8, 32), _f32).at[0:3].set(p['ete'])
    vvw1 = p['vv_w1']
    vvb1 = p['vv_b1'].reshape(1, 64)
    vvw2r = p['vv_w2'].reshape(1, 64)
    vvb2b = jnp.broadcast_to(p['vv_b2'].reshape(1, 1), (1, 128))
    aw2p = jnp.zeros((128, 32), _f32).at[:, 0:ATOM_NF].set(p['ad_w2'])
    ab2p = jnp.zeros((1, 32), _f32).at[0, 0:ATOM_NF].set(p['ad_b2'])
    rw2p = jnp.zeros((128, 32), _f32).at[:, 0:RES_NF].set(p['rd_w2'])
    rb2p = jnp.zeros((1, 32), _f32).at[0, 0:RES_NF].set(p['rd_b2'])

    h0 = _prologue(xh_all, w1c, ab1, rb1, p['ae_w2'], ab2, p['re_w2'], rb2,
                   win0, win1, tb)
    dxr, dyr, dzr, d2r = _geom(px, py, pz, src, dst)
    ux, uy, uz, filtT = _features(
        dxr.reshape(EP // 128, 128), dyr.reshape(EP // 128, 128),
        dzr.reshape(EP // 128, 128), d2r.reshape(EP // 128, 128),
        et2d, p['mp_wrbf'], ete8)
    ux = ux.reshape(EP); uy = uy.reshape(EP); uz = uz.reshape(EP)

    vec3 = jnp.zeros((3, NP, 128), _f32)
    sd5 = jnp.zeros((8, EP), _i32)
    sd5 = sd5.at[0].set(src).at[1].set(dst)
    sd5 = sd5.at[2].set(lax.bitcast_convert_type(ux, _i32))
    sd5 = sd5.at[3].set(lax.bitcast_convert_type(uy, _i32))
    sd5 = sd5.at[4].set(lax.bitcast_convert_type(uz, _i32))

    for l in range(NLAYERS):
        hw = _hw(h0, p['mp_wh'][l])
        sco = _msg(hw, filtT[l], sd5)
        h0, vec3 = _update(h0, sco.reshape(4, NP, 128), vec3, p['mp_wvec'][l])

    velp, sums = _vel(vec3, vvw1, vvb1, vvw2r, vvb2b, batch)
    res = _decode(velp, sums, batch, h0,
                  p['ad_w1'], p['ad_b1'].reshape(1, 128), aw2p, ab2p,
                  p['rd_w1'], p['rd_b1'].reshape(1, 128), rw2p, rb2p)
    atoms_output = res[:N_AT, 0:3 + ATOM_NF]
    residues_output = res[N_AT:NN, 0:3 + RES_NF]
    return (atoms_output, residues_output)


# speedup vs baseline: 1.8408x; 1.0054x over previous
import functools
import numpy as np
import jax
import jax.numpy as jnp
from jax import lax
from jax.experimental import pallas as pl
from jax.experimental.pallas import tpu as pltpu
from jax.experimental.pallas import tpu_sc as plsc

ATOM_NF = 16; RES_NF = 21; HID = 128; NRBF = 32; CUTOFF = 5.0
NLAYERS = 4; NGRAPH = 32
N_AT = 5000; N_RES = 5000; NN = 10000; NE = 160000
NP = 10240
EP = 163840
NCORE = 2; NSUB = 16
KB = 128
ET = EP // NSUB
NBLK = ET // KB
RPT = NP // NSUB
EPT = EP // (NCORE * NSUB)
RB = 256
NRB = NP // RB

_f32 = jnp.float32
_i32 = jnp.int32

_ALPHA = 5.0 / CUTOFF
_MEANS = np.linspace(float(np.exp(-CUTOFF)), 1.0, NRBF).astype(np.float32)
_BETA = float(((2.0 / NRBF) * (1.0 - np.exp(-CUTOFF))) ** -2)


def _silu(x):
    return x * (1.0 / (1.0 + jnp.exp(-x)))


def _prologue_body(xh_ref, w1_ref, ab1_ref, rb1_ref, w2a_ref, ab2_ref,
                   w2r_ref, rb2_ref, win0_ref, win1_ref, t_ref, h0_ref):
    i = pl.program_id(0)
    x = xh_ref[...]
    rows = i * RB + lax.broadcasted_iota(_i32, (RB, 1), 0)
    is_atom = rows < N_AT
    b1 = jnp.where(is_atom, ab1_ref[...], rb1_ref[...])
    h1 = _silu(jnp.dot(x, w1_ref[...], preferred_element_type=_f32) + b1)
    h2a = jnp.dot(h1, w2a_ref[...], preferred_element_type=_f32) + ab2_ref[...]
    h2r = jnp.dot(h1, w2r_ref[...], preferred_element_type=_f32) + rb2_ref[...]
    h2 = jnp.where(is_atom, h2a, h2r)
    h0 = jnp.dot(h2, win0_ref[...], preferred_element_type=_f32)
    h0_ref[...] = h0 + t_ref[0, 0] * win1_ref[...]


def _prologue(xh_all, w1c, ab1, rb1, w2a, ab2, w2r, rb2, win0, win1, tb):
    whole = lambda shape: pl.BlockSpec(shape, lambda i: (0,) * len(shape))
    return pl.pallas_call(
        _prologue_body,
        grid=(NRB,),
        in_specs=[pl.BlockSpec((RB, 128), lambda i: (i, 0)),
                  whole((128, 128)), whole((1, 128)), whole((1, 128)),
                  whole((128, 128)), whole((1, 128)),
                  whole((128, 128)), whole((1, 128)),
                  whole((128, 128)), whole((1, 128)), whole((1, 128))],
        out_specs=pl.BlockSpec((RB, 128), lambda i: (i, 0)),
        out_shape=jax.ShapeDtypeStruct((NP, 128), _f32),
    )(xh_all, w1c, ab1, rb1, w2a, ab2, w2r, rb2, win0, win1, tb)


def _geom_body(px_hbm, py_hbm, pz_hbm, src_hbm, dst_hbm,
               dx_hbm, dy_hbm, dz_hbm, d2_hbm,
               xs, ys, zs, sbuf, dbuf, ox, oy, oz, o2):
    cid = lax.axis_index("c")
    sid = lax.axis_index("s")
    wid = cid * NSUB + sid
    base = wid * EPT
    pltpu.sync_copy(px_hbm, xs)
    pltpu.sync_copy(py_hbm, ys)
    pltpu.sync_copy(pz_hbm, zs)
    pltpu.sync_copy(src_hbm.at[pl.ds(base, EPT)], sbuf)
    pltpu.sync_copy(dst_hbm.at[pl.ds(base, EPT)], dbuf)

    def body(g, _):
        s16 = sbuf[pl.ds(g * 16, 16)]
        d16 = dbuf[pl.ds(g * 16, 16)]
        dxv = plsc.load_gather(xs, [d16]) - plsc.load_gather(xs, [s16])
        dyv = plsc.load_gather(ys, [d16]) - plsc.load_gather(ys, [s16])
        dzv = plsc.load_gather(zs, [d16]) - plsc.load_gather(zs, [s16])
        d2v = dxv * dxv + dyv * dyv + dzv * dzv
        ox[pl.ds(g * 16, 16)] = dxv
        oy[pl.ds(g * 16, 16)] = dyv
        oz[pl.ds(g * 16, 16)] = dzv
        o2[pl.ds(g * 16, 16)] = d2v
        return 0

    lax.fori_loop(0, EPT // 16, body, 0)
    pltpu.sync_copy(ox, dx_hbm.at[pl.ds(base, EPT)])
    pltpu.sync_copy(oy, dy_hbm.at[pl.ds(base, EPT)])
    pltpu.sync_copy(oz, dz_hbm.at[pl.ds(base, EPT)])
    pltpu.sync_copy(o2, d2_hbm.at[pl.ds(base, EPT)])


def _geom(px, py, pz, src, dst):
    mesh = plsc.VectorSubcoreMesh(core_axis_name="c", subcore_axis_name="s", num_cores=NCORE, num_subcores=NSUB)
    out = jax.ShapeDtypeStruct((EP,), _f32)
    fn = pl.kernel(
        _geom_body,
        out_type=(out, out, out, out),
        mesh=mesh,
        compiler_params=pltpu.CompilerParams(needs_layout_passes=False),
        scratch_types=[pltpu.VMEM((NP,), _f32)] * 3
        + [pltpu.VMEM((EPT,), _i32)] * 2
        + [pltpu.VMEM((EPT,), _f32)] * 4,
    )
    return fn(px, py, pz, src, dst)


def _feat_body(dx_ref, dy_ref, dz_ref, d2_ref, et_ref, wrbf_ref, ete_ref,
               ux_ref, uy_ref, uz_ref, filt_ref):
    i = pl.program_id(0)
    d2 = d2_ref[...]
    dist = jnp.sqrt(d2 + 1e-12)
    inv = 1.0 / (dist + 1e-8)
    ux_ref[...] = dx_ref[...] * inv
    uy_ref[...] = dy_ref[...] * inv
    uz_ref[...] = dz_ref[...] * inv
    ed = jnp.exp(-_ALPHA * dist)
    env = 0.5 * (jnp.cos(jnp.pi * jnp.clip(dist, 0.0, CUTOFF) / CUTOFF) + 1.0)
    m0 = float(np.exp(-CUTOFF))
    means = (m0 + lax.broadcasted_iota(_i32, (NRBF, 1), 0).astype(_f32)
             * ((1.0 - m0) / (NRBF - 1)))
    eidx = (i * 1024 + lax.broadcasted_iota(_i32, (8, 128), 0) * 128
            + lax.broadcasted_iota(_i32, (8, 128), 1))
    valid = eidx < NE
    envm = jnp.where(valid, env, 0.0)
    et = jnp.where(valid, et_ref[...], -1)
    g3 = lax.broadcasted_iota(_i32, (3, 1), 0)
    parts = []
    for r in range(8):
        edr = ed[r:r + 1]
        rbf = jnp.exp(-_BETA * (edr - means) ** 2) * envm[r:r + 1]
        ohf = (g3 == et[r:r + 1]).astype(_f32)
        parts.append(jnp.concatenate(
            [rbf, ohf, jnp.zeros((13, 128), _f32)], axis=0))
    feat = jnp.concatenate(parts, axis=1)
    for l in range(NLAYERS):
        wr = wrbf_ref[l]
        etew = jnp.dot(ete_ref[...], wr, preferred_element_type=_f32)
        w = jnp.concatenate([wr, etew[0:3], jnp.zeros((13, 128), _f32)], axis=0)
        ft = lax.dot_general(w, feat, (((0,), (0,)), ((), ())),
                             preferred_element_type=_f32)
        filt_ref[l] = _silu(ft)


def _features(dxr, dyr, dzr, d2r, et2d, wrbf, ete8):
    row = pl.BlockSpec((8, 128), lambda i: (i, 0))
    outr = jax.ShapeDtypeStruct((EP // 128, 128), _f32)
    return pl.pallas_call(
        _feat_body,
        grid=(EP // 1024,),
        in_specs=[row, row, row, row, row,
                  pl.BlockSpec((NLAYERS, NRBF, 128), lambda i: (0, 0, 0)),
                  pl.BlockSpec((8, 32), lambda i: (0, 0))],
        out_specs=[row, row, row,
                   pl.BlockSpec((NLAYERS, 128, 1024), lambda i: (0, 0, i))],
        out_shape=[outr, outr, outr,
                   jax.ShapeDtypeStruct((NLAYERS, 128, EP), _f32)],
    )(dxr, dyr, dzr, d2r, et2d, wrbf, ete8)


def _hw_body(h0_ref, wh_ref, out_ref):
    out_ref[...] = jnp.dot(h0_ref[...], wh_ref[...],
                           preferred_element_type=_f32)


def _hw(h0, wh):
    return pl.pallas_call(
        _hw_body,
        grid=(NRB,),
        in_specs=[pl.BlockSpec((RB, 128), lambda i: (i, 0)),
                  pl.BlockSpec((128, 128), lambda i: (0, 0))],
        out_specs=pl.BlockSpec((RB, 128), lambda i: (i, 0)),
        out_shape=jax.ShapeDtypeStruct((NP, 128), _f32),
    )(h0, wh)


def _msg_body(hw_hbm, filt_hbm, sd5_hbm, out_hbm,
              acc, pbuf, fbuf, hbufa, hbufb, hbufc, hbufd, obufa, obufb,
              dq0, dq1, dq2, dq3, dq4, dq5, dq6, dq7,
              ssem, gsem, osem):
    cid = lax.axis_index("c")
    sid = lax.axis_index("s")
    ebase = sid * ET
    dqs = [dq0, dq1, dq2, dq3, dq4, dq5, dq6, dq7]

    def zero_obuf(r, _):
        for q in range(8):
            obufa[r, pl.ds(q * 16, 16)] = jnp.zeros((16,), _f32)
        return 0

    def drain_scatter(slot):
        ob = obufa if slot == 0 else obufb
        pltpu.make_async_copy(ob, acc.at[dqs[slot]], osem).wait()

    for p in range(2):
        c = cid * 2 + p
        lax.fori_loop(0, 16, zero_obuf, 0)

        def zero_acc(q, _):
            pltpu.sync_copy(obufa, acc.at[pl.ds(sid * RPT + q * 16, 16)])
            return 0
        lax.fori_loop(0, RPT // 16, zero_acc, 0)
        plsc.subcore_barrier()

        pltpu.async_copy(sd5_hbm.at[:, pl.ds(ebase, KB)], pbuf.at[0], ssem)
        pltpu.async_copy(filt_hbm.at[pl.ds(c * 32, 32), pl.ds(ebase, KB)],
                         fbuf.at[0], ssem)

        def block(b, _):
            sel = lax.rem(b, 2)
            seln = 1 - sel
            pltpu.make_async_copy(
                sd5_hbm.at[:, pl.ds(0, KB)], pbuf.at[sel], ssem).wait()
            pltpu.make_async_copy(
                filt_hbm.at[pl.ds(0, 32), pl.ds(0, KB)],
                fbuf.at[sel], ssem).wait()

            @pl.when(b + 1 < NBLK)
            def _():
                e1 = ebase + (b + 1) * KB
                pltpu.async_copy(sd5_hbm.at[:, pl.ds(e1, KB)],
                                 pbuf.at[seln], ssem)
                pltpu.async_copy(
                    filt_hbm.at[pl.ds(c * 32, 32), pl.ds(e1, KB)],
                    fbuf.at[seln], ssem)

            @pl.when(b > 0)
            def _():
                drain_scatter(0)
                drain_scatter(1)

            for o in range(8):
                dqs[o][...] = pbuf[sel, 1, pl.ds(o * 16, 16)]
            hbs = [hbufa, hbufb, hbufc, hbufd]
            gds = [pltpu.async_copy(
                       hw_hbm.at[pbuf.at[sel, 0, pl.ds(j * 16, 16)]],
                       hbs[j], gsem) for j in range(4)]
            hbase = jnp.full((16,), 32, _i32) * c
            el = lax.iota(_i32, 16)
            for o in range(8):
                gds[o].wait()
                hb = hbs[o % 4]
                ob = obufa if o % 2 == 0 else obufb
                if o >= 2:
                    drain_scatter(o % 2)
                be = o * 16
                vx = plsc.bitcast(pbuf[sel, 2, pl.ds(be, 16)], _f32)
                vy = plsc.bitcast(pbuf[sel, 3, pl.ds(be, 16)], _f32)
                vz = plsc.bitcast(pbuf[sel, 4, pl.ds(be, 16)], _f32)
                for h in range(32):
                    f = fbuf[sel, h, pl.ds(be, 16)]
                    w = plsc.load_gather(hb, [el, hbase + h])
                    m = f * w
                    plsc.store_scatter(
                        ob, [el, jnp.full((16,), h, _i32)], m)
                    plsc.store_scatter(
                        ob, [el, jnp.full((16,), 32 + h, _i32)], m * vx)
                    plsc.store_scatter(
                        ob, [el, jnp.full((16,), 64 + h, _i32)], m * vy)
                    plsc.store_scatter(
                        ob, [el, jnp.full((16,), 96 + h, _i32)], m * vz)
                if o < 4:
                    gds.append(pltpu.async_copy(
                        hw_hbm.at[pbuf.at[sel, 0, pl.ds((o + 4) * 16, 16)]],
                        hbs[o % 4], gsem))
                pltpu.async_copy(ob, acc.at[dqs[o]], osem, add=True)
            return 0

        lax.fori_loop(0, NBLK, block, 0)
        drain_scatter(0)
        drain_scatter(1)
        plsc.subcore_barrier()
        pltpu.sync_copy(acc.at[pl.ds(sid * RPT, RPT)],
                        out_hbm.at[pl.ds(c * NP + sid * RPT, RPT)])
        plsc.subcore_barrier()


def _msg(hw, filt_l, sd5):
    mesh = plsc.VectorSubcoreMesh(core_axis_name="c", subcore_axis_name="s", num_cores=NCORE, num_subcores=NSUB)
    fn = pl.kernel(
        _msg_body,
        out_type=jax.ShapeDtypeStruct((4 * NP, 128), _f32),
        mesh=mesh,
        compiler_params=pltpu.CompilerParams(needs_layout_passes=False),
        scratch_types=[
            pltpu.VMEM_SHARED((NP, 128), _f32),
            pltpu.VMEM((2, 8, KB), _i32),
            pltpu.VMEM((2, 32, KB), _f32),
            pltpu.VMEM((16, 128), _f32),
            pltpu.VMEM((16, 128), _f32),
            pltpu.VMEM((16, 128), _f32),
            pltpu.VMEM((16, 128), _f32),
            pltpu.VMEM((16, 128), _f32),
            pltpu.VMEM((16, 128), _f32),
            pltpu.VMEM((16,), _i32),
            pltpu.VMEM((16,), _i32),
            pltpu.VMEM((16,), _i32),
            pltpu.VMEM((16,), _i32),
            pltpu.VMEM((16,), _i32),
            pltpu.VMEM((16,), _i32),
            pltpu.VMEM((16,), _i32),
            pltpu.VMEM((16,), _i32),
            pltpu.SemaphoreType.DMA,
            pltpu.SemaphoreType.DMA,
            pltpu.SemaphoreType.DMA,
        ],
    )
    return fn(hw, filt_l, sd5)


def _update_body(h0_ref, sco_ref, vec_ref, wv_ref, h0o_ref, veco_ref):
    s = sco_ref[...]
    agg = jnp.concatenate([s[k, :, 0:32] for k in range(4)], axis=1)
    h0o_ref[...] = h0_ref[...] + _silu(agg)
    wv = wv_ref[...]
    for ci in range(3):
        c0 = 32 * (ci + 1)
        d = jnp.concatenate([s[k, :, c0:c0 + 32] for k in range(4)], axis=1)
        veco_ref[ci] = jnp.dot(vec_ref[ci] + d, wv, preferred_element_type=_f32)


def _update(h0, sco4, vec3, wv):
    return pl.pallas_call(
        _update_body,
        grid=(NRB,),
        in_specs=[pl.BlockSpec((RB, 128), lambda i: (i, 0)),
                  pl.BlockSpec((4, RB, 128), lambda i: (0, i, 0)),
                  pl.BlockSpec((3, RB, 128), lambda i: (0, i, 0)),
                  pl.BlockSpec((128, 128), lambda i: (0, 0))],
        out_specs=[pl.BlockSpec((RB, 128), lambda i: (i, 0)),
                   pl.BlockSpec((3, RB, 128), lambda i: (0, i, 0))],
        out_shape=[jax.ShapeDtypeStruct((NP, 128), _f32),
                   jax.ShapeDtypeStruct((3, NP, 128), _f32)],
    )(h0, sco4, vec3, wv)


def _vel_body(vec_ref, w1_ref, b1_ref, w2_ref, b2_ref, bt_ref,
              velp_ref, sums_ref):
    i = pl.program_id(0)
    cols = []
    w2row = w2_ref[...]
    for ci in range(3):
        sv = _silu(jnp.dot(vec_ref[ci], w1_ref[...],
                           preferred_element_type=_f32) + b1_ref[...])
        r = jnp.sum(sv * w2row, axis=1, keepdims=True) + b2_ref[0, 0]
        cols.append(r)
    rows = i * RB + lax.broadcasted_iota(_i32, (RB, 1), 0)
    ones = jnp.where(rows < NN, 1.0, 0.0)
    velp = jnp.concatenate(cols + [ones, jnp.zeros((RB, 124), _f32)], axis=1)
    velp_ref[...] = velp
    bt = bt_ref[...]
    oh = (lax.broadcasted_iota(_i32, (NGRAPH, 1), 0) == bt).astype(_f32)
    contrib = jnp.dot(oh, velp, preferred_element_type=_f32)

    @pl.when(i == 0)
    def _():
        sums_ref[...] = jnp.zeros((NGRAPH, 128), _f32)

    sums_ref[...] += contrib


def _vel(vec3, vvw1, vvb1, vvw2r, vvb2b, batch):
    return pl.pallas_call(
        _vel_body,
        grid=(NRB,),
        in_specs=[pl.BlockSpec((3, RB, 128), lambda i: (0, i, 0)),
                  pl.BlockSpec((128, 64), lambda i: (0, 0)),
                  pl.BlockSpec((1, 64), lambda i: (0, 0)),
                  pl.BlockSpec((1, 64), lambda i: (0, 0)),
                  pl.BlockSpec((1, 128), lambda i: (0, 0)),
                  pl.BlockSpec((RB,), lambda i: (i,))],
        out_specs=[pl.BlockSpec((RB, 128), lambda i: (i, 0)),
                   pl.BlockSpec((NGRAPH, 128), lambda i: (0, 0))],
        out_shape=[jax.ShapeDtypeStruct((NP, 128), _f32),
                   jax.ShapeDtypeStruct((NGRAPH, 128), _f32)],
    )(vec3, vvw1, vvb1, vvw2r, vvb2b, batch)


def _dec_body(velp_ref, sums_ref, bt_ref, h0_ref, aw1_ref, ab1_ref, aw2_ref,
              ab2_ref, rw1_ref, rb1_ref, rw2_ref, rb2_ref, out_ref):
    i = pl.program_id(0)
    s = sums_ref[...]
    cnt = s[:, 3:4]
    mean = s * (1.0 / jnp.maximum(cnt, 1.0))
    bt = bt_ref[...]
    oh = (lax.broadcasted_iota(_i32, (NGRAPH, 1), 0) == bt).astype(_f32)
    meanrows = lax.dot_general(oh, mean, (((0,), (0,)), ((), ())),
                               preferred_element_type=_f32)
    vel = velp_ref[...] - meanrows
    h = h0_ref[...]
    ha = jnp.dot(_silu(jnp.dot(h, aw1_ref[...], preferred_element_type=_f32)
                       + ab1_ref[...]), aw2_ref[...],
                 preferred_element_type=_f32) + ab2_ref[...]
    hr = jnp.dot(_silu(jnp.dot(h, rw1_ref[...], preferred_element_type=_f32)
                       + rb1_ref[...]), rw2_ref[...],
                 preferred_element_type=_f32) + rb2_ref[...]
    rows = i * RB + lax.broadcasted_iota(_i32, (RB, 1), 0)
    hf = jnp.where(rows < N_AT, ha, hr)
    out_ref[...] = jnp.concatenate(
        [vel[:, 0:3], hf[:, 0:21], jnp.zeros((RB, 8), _f32)], axis=1)


def _decode(velp, sums, batch, h0, aw1, ab1, aw2p, ab2p, rw1, rb1, rw2p, rb2p):
    whole = lambda shape: pl.BlockSpec(shape, lambda i: (0,) * len(shape))
    return pl.pallas_call(
        _dec_body,
        grid=(NRB,),
        in_specs=[pl.BlockSpec((RB, 128), lambda i: (i, 0)),
                  whole((NGRAPH, 128)),
                  pl.BlockSpec((RB,), lambda i: (i,)),
                  pl.BlockSpec((RB, 128), lambda i: (i, 0)),
                  whole((128, 128)), whole((1, 128)), whole((128, 32)),
                  whole((1, 32)),
                  whole((128, 128)), whole((1, 128)), whole((128, 32)),
                  whole((1, 32))],
        out_specs=pl.BlockSpec((RB, 32), lambda i: (i, 0)),
        out_shape=jax.ShapeDtypeStruct((NP, 32), _f32),
    )(velp, sums, batch, h0, aw1, ab1, aw2p, ab2p, rw1, rb1, rw2p, rb2p)


def kernel(xh_atoms, xh_residues, t, mask_atoms, mask_residues,
           edge_index, edge_types, params):
    p = params
    xa = xh_atoms.astype(_f32)
    xr = xh_residues.astype(_f32)

    xh_all = jnp.zeros((NP, 128), _f32)
    xh_all = xh_all.at[:N_AT, 0:ATOM_NF].set(xa[:, 3:])
    xh_all = xh_all.at[N_AT:NN, ATOM_NF:ATOM_NF + RES_NF].set(xr[:, 3:])
    pcat = jnp.concatenate([xa[:, :3], xr[:, :3],
                            jnp.zeros((NP - NN, 3), _f32)], axis=0)
    px = pcat[:, 0]; py = pcat[:, 1]; pz = pcat[:, 2]

    src = jnp.zeros((EP,), _i32).at[:NE].set(edge_index[0].astype(_i32))
    dst = jnp.zeros((EP,), _i32).at[:NE].set(edge_index[1].astype(_i32))
    et2d = (jnp.zeros((EP,), _i32).at[:NE].set(edge_types.astype(_i32))
            .reshape(EP // 128, 128))
    batch = (jnp.full((NP,), -1, _i32)
             .at[:N_AT].set(mask_atoms.astype(_i32))
             .at[N_AT:NN].set(mask_residues.astype(_i32)))

    w1c = (jnp.zeros((128, 128), _f32)
           .at[0:ATOM_NF].set(p['ae_w1'])
           .at[ATOM_NF:ATOM_NF + RES_NF].set(p['re_w1']))
    ab1 = p['ae_b1'].reshape(1, 128); rb1 = p['re_b1'].reshape(1, 128)
    ab2 = p['ae_b2'].reshape(1, 128); rb2 = p['re_b2'].reshape(1, 128)
    win0 = p['win'][:128]
    win1 = p['win'][128:129]
    tb = jnp.broadcast_to(t.reshape(1, 1).astype(_f32), (1, 128))
    ete8 = jnp.zeros((8, 32), _f32).at[0:3].set(p['ete'])
    vvw1 = p['vv_w1']
    vvb1 = p['vv_b1'].reshape(1, 64)
    vvw2r = p['vv_w2'].reshape(1, 64)
    vvb2b = jnp.broadcast_to(p['vv_b2'].reshape(1, 1), (1, 128))
    aw2p = jnp.zeros((128, 32), _f32).at[:, 0:ATOM_NF].set(p['ad_w2'])
    ab2p = jnp.zeros((1, 32), _f32).at[0, 0:ATOM_NF].set(p['ad_b2'])
    rw2p = jnp.zeros((128, 32), _f32).at[:, 0:RES_NF].set(p['rd_w2'])
    rb2p = jnp.zeros((1, 32), _f32).at[0, 0:RES_NF].set(p['rd_b2'])

    h0 = _prologue(xh_all, w1c, ab1, rb1, p['ae_w2'], ab2, p['re_w2'], rb2,
                   win0, win1, tb)
    dxr, dyr, dzr, d2r = _geom(px, py, pz, src, dst)
    ux, uy, uz, filtT = _features(
        dxr.reshape(EP // 128, 128), dyr.reshape(EP // 128, 128),
        dzr.reshape(EP // 128, 128), d2r.reshape(EP // 128, 128),
        et2d, p['mp_wrbf'], ete8)
    ux = ux.reshape(EP); uy = uy.reshape(EP); uz = uz.reshape(EP)

    vec3 = jnp.zeros((3, NP, 128), _f32)
    sd5 = jnp.zeros((8, EP), _i32)
    sd5 = sd5.at[0].set(src).at[1].set(dst)
    sd5 = sd5.at[2].set(lax.bitcast_convert_type(ux, _i32))
    sd5 = sd5.at[3].set(lax.bitcast_convert_type(uy, _i32))
    sd5 = sd5.at[4].set(lax.bitcast_convert_type(uz, _i32))

    for l in range(NLAYERS):
        hw = _hw(h0, p['mp_wh'][l])
        sco = _msg(hw, filtT[l], sd5)
        h0, vec3 = _update(h0, sco.reshape(4, NP, 128), vec3, p['mp_wvec'][l])

    velp, sums = _vel(vec3, vvw1, vvb1, vvw2r, vvb2b, batch)
    res = _decode(velp, sums, batch, h0,
                  p['ad_w1'], p['ad_b1'].reshape(1, 128), aw2p, ab2p,
                  p['rd_w1'], p['rd_b1'].reshape(1, 128), rw2p, rb2p)
    atoms_output = res[:N_AT, 0:3 + ATOM_NF]
    residues_output = res[N_AT:NN, 0:3 + RES_NF]
    return (atoms_output, residues_output)
